# Initial kernel scaffold; baseline (speedup 1.0000x reference)
import functools

import jax
import jax.numpy as jnp
from jax import lax
from jax.experimental import pallas as pl
from jax.experimental.pallas import tpu as pltpu
from jax.experimental.pallas import tpu_sc as plsc

NLANE = 16
NTILE = 16
NCORE = 2
NW = NCORE * NTILE


def _mesh():
    return plsc.VectorSubcoreMesh(core_axis_name="c", subcore_axis_name="s")


@functools.lru_cache(maxsize=None)
def _make_sc_norm(E, NPAD):
    SLICE = NPAD // NTILE
    ED = E // NTILE
    ECH = 2000
    EN = E // NW
    NCH = 2000
    assert ED % ECH == 0 and EN % NCH == 0 and SLICE % NLANE == 0

    def body(src_hbm, dst_hbm, ew_hbm, norm_hbm, inv_hbm,
             acc, dis_loc, sb, db, wb, nb, psum, dslice, islice,
             sp_part, sp_dis):
        cid = lax.axis_index("c")
        sid = lax.axis_index("s")
        iota = lax.iota(jnp.int32, NLANE)
        plane = jnp.bitwise_and(iota, 7)
        mlow = iota < 8
        mhigh = jnp.logical_not(mlow)
        fzero = jnp.zeros((NLANE,), jnp.float32)

        def zloop(i, c):
            acc[pl.ds(i * NLANE, NLANE)] = fzero
            return c
        lax.fori_loop(0, (8 * NPAD) // NLANE, zloop, 0)

        def deg_chunk(k, c):
            base = sid * ED + k * ECH
            pltpu.sync_copy(dst_hbm.at[pl.ds(base, ECH)], db)
            pltpu.sync_copy(ew_hbm.at[pl.ds(base, ECH)], wb)

            def deg16(i, cc):
                d16 = db[pl.ds(i * NLANE, NLANE)]
                w16 = wb[pl.ds(i * NLANE, NLANE)]
                fidx = plane * NPAD + d16
                plsc.addupdate_scatter(acc, [fidx], w16, mask=mlow)
                plsc.addupdate_scatter(acc, [fidx], w16, mask=mhigh)
                return cc
            lax.fori_loop(0, ECH // NLANE, deg16, 0)
            return c
        lax.fori_loop(0, ED // ECH, deg_chunk, 0)

        def red(i, c):
            s = acc[pl.ds(i * NLANE, NLANE)]
            for p in range(1, 8):
                s = s + acc[pl.ds(p * NPAD + i * NLANE, NLANE)]
            dis_loc[pl.ds(i * NLANE, NLANE)] = s
            return c
        lax.fori_loop(0, NPAD // NLANE, red, 0)

        pltpu.sync_copy(dis_loc, sp_part.at[pl.ds(sid * NPAD, NPAD)])
        plsc.subcore_barrier()

        for r in range(NTILE):
            pltpu.sync_copy(sp_part.at[pl.ds(r * NPAD + sid * SLICE, SLICE)],
                            psum.at[pl.ds(r * SLICE, SLICE)])

        magic = jnp.full((NLANE,), 0x5F3759DF, jnp.int32)

        def disloop(i, c):
            s = psum[pl.ds(i * NLANE, NLANE)]
            for r in range(1, NTILE):
                s = s + psum[pl.ds(r * SLICE + i * NLANE, NLANE)]
            d = s + 1.0
            ibits = plsc.bitcast(d, jnp.int32)
            y = plsc.bitcast(magic - lax.shift_right_logical(ibits, 1),
                             jnp.float32)
            for _ in range(3):
                y = y * (1.5 - 0.5 * d * y * y)
            dslice[pl.ds(i * NLANE, NLANE)] = y
            islice[pl.ds(i * NLANE, NLANE)] = y * y
            return c
        lax.fori_loop(0, SLICE // NLANE, disloop, 0)

        pltpu.sync_copy(dslice, sp_dis.at[pl.ds(sid * SLICE, SLICE)])

        @pl.when(cid == 0)
        def _():
            pltpu.sync_copy(islice, inv_hbm.at[pl.ds(sid * SLICE, SLICE)])

        plsc.subcore_barrier()
        pltpu.sync_copy(sp_dis, dis_loc)

        wid = sid * NCORE + cid

        def nchunk(k, c):
            base = wid * EN + k * NCH
            pltpu.sync_copy(src_hbm.at[pl.ds(base, NCH)], sb)
            pltpu.sync_copy(dst_hbm.at[pl.ds(base, NCH)], db)
            pltpu.sync_copy(ew_hbm.at[pl.ds(base, NCH)], wb)

            def n16(i, cc):
                s16 = sb[pl.ds(i * NLANE, NLANE)]
                d16 = db[pl.ds(i * NLANE, NLANE)]
                w16 = wb[pl.ds(i * NLANE, NLANE)]
                a = plsc.load_gather(dis_loc, [s16])
                b = plsc.load_gather(dis_loc, [d16])
                nb[pl.ds(i * NLANE, NLANE)] = a * w16 * b
                return cc
            lax.fori_loop(0, NCH // NLANE, n16, 0)
            pltpu.sync_copy(nb, norm_hbm.at[pl.ds(base, NCH)])
            return c
        lax.fori_loop(0, EN // NCH, nchunk, 0)

    return pl.kernel(
        body,
        out_type=[jax.ShapeDtypeStruct((E,), jnp.float32),
                  jax.ShapeDtypeStruct((NPAD,), jnp.float32)],
        mesh=_mesh(),
        scratch_types=[
            pltpu.VMEM((8 * NPAD,), jnp.float32),
            pltpu.VMEM((NPAD,), jnp.float32),
            pltpu.VMEM((NCH,), jnp.int32),
            pltpu.VMEM((NCH,), jnp.int32),
            pltpu.VMEM((NCH,), jnp.float32),
            pltpu.VMEM((NCH,), jnp.float32),
            pltpu.VMEM((NTILE * SLICE,), jnp.float32),
            pltpu.VMEM((SLICE,), jnp.float32),
            pltpu.VMEM((SLICE,), jnp.float32),
            pltpu.VMEM_SHARED((NTILE * NPAD,), jnp.float32),
            pltpu.VMEM_SHARED((NPAD,), jnp.float32),
        ],
    )


@functools.lru_cache(maxsize=None)
def _make_sc_agg(E, NPAD, P):
    NG = NW // P
    EG = E // NG
    CH = 2000
    SLEN = 80
    NS = CH // SLEN
    NC = EG // CH
    assert EG % CH == 0 and CH % SLEN == 0 and SLEN % 8 == 0

    def body(tab_hbm, src_hbm, dst_hbm, norm_hbm, out_hbm,
             acc, rows, sb, db, nbuf, sem):
        cid = lax.axis_index("c")
        sid = lax.axis_index("s")
        wid = sid * NCORE + cid
        g = wid // P
        t = wid - g * P
        iota = lax.iota(jnp.int32, NLANE)
        col = jnp.bitwise_and(iota, 7)
        c01 = lax.shift_right_logical(iota, 3)
        mlow = iota < 8
        mhigh = jnp.logical_not(mlow)
        fzero = jnp.zeros((NLANE,), jnp.float32)

        def zloop(i, c):
            acc[pl.ds(i * NLANE, NLANE)] = fzero
            return c
        lax.fori_loop(0, (8 * NPAD) // NLANE, zloop, 0)

        toff = jnp.full((NLANE,), 1, jnp.int32) * (t * NPAD)

        def chunk(k, c):
            base = g * EG + k * CH
            pltpu.sync_copy(src_hbm.at[pl.ds(base, CH)], sb)
            pltpu.sync_copy(dst_hbm.at[pl.ds(base, CH)], db)
            pltpu.sync_copy(norm_hbm.at[pl.ds(base, CH)], nbuf)

            def adj(i, cc):
                sb[pl.ds(i * NLANE, NLANE)] = (
                    sb[pl.ds(i * NLANE, NLANE)] + toff)
                return cc
            lax.fori_loop(0, CH // NLANE, adj, 0)

            for j in range(NS):
                pltpu.async_copy(
                    tab_hbm.at[sb.at[pl.ds(j * SLEN, SLEN)]],
                    rows.at[pl.ds(j * SLEN, SLEN)], sem)
            for j in range(NS):
                pltpu.make_async_copy(
                    tab_hbm.at[sb.at[pl.ds(j * SLEN, SLEN)]],
                    rows.at[pl.ds(j * SLEN, SLEN)], sem).wait()

            def pair(p, cc):
                psel = jnp.full((NLANE,), 2, jnp.int32) * p + c01
                dstp = plsc.load_gather(db, [psel])
                normp = plsc.load_gather(nbuf, [psel])
                r16 = plsc.load_gather(rows, [psel, col])
                val = r16 * normp
                fidx = dstp * 8 + col
                plsc.addupdate_scatter(acc, [fidx], val, mask=mlow)
                plsc.addupdate_scatter(acc, [fidx], val, mask=mhigh)
                return cc
            lax.fori_loop(0, CH // 2, pair, 0)
            return c
        lax.fori_loop(0, NC, chunk, 0)

        pltpu.sync_copy(acc, out_hbm.at[pl.ds(wid * 8 * NPAD, 8 * NPAD)])

    return pl.kernel(
        body,
        out_type=jax.ShapeDtypeStruct((NW * 8 * NPAD,), jnp.float32),
        mesh=_mesh(),
        scratch_types=[
            pltpu.VMEM((8 * NPAD,), jnp.float32),
            pltpu.VMEM((CH, 8), jnp.float32),
            pltpu.VMEM((CH,), jnp.int32),
            pltpu.VMEM((CH,), jnp.int32),
            pltpu.VMEM((CH,), jnp.float32),
            pltpu.SemaphoreType.DMA,
        ],
    )


@functools.lru_cache(maxsize=None)
def _make_tc_pre(N, NPAD, DIN, F):
    P = F // 8

    def body(x_ref, w_ref, out_ref):
        h = jnp.dot(x_ref[...], w_ref[...],
                    preferred_element_type=jnp.float32)
        for g in range(P):
            out_ref[g, :N] = h[:, g * 8:(g + 1) * 8]
            out_ref[g, N:] = jnp.zeros((NPAD - N, 8), jnp.float32)

    return pl.pallas_call(
        body,
        out_shape=jax.ShapeDtypeStruct((P, NPAD, 8), jnp.float32))


def _bn(h, g, b):
    m = jnp.mean(h, axis=0, keepdims=True)
    v = jnp.mean((h - m) * (h - m), axis=0, keepdims=True)
    return (h - m) * lax.rsqrt(v + 1e-5) * g + b


@functools.lru_cache(maxsize=None)
def _make_tc_mid(N, NPAD, F1, F2):
    P1, P2, NG1 = F1 // 8, F2 // 8, NW // (F1 // 8)

    def body(part_ref, ht_ref, inv_ref, b_ref, g_ref, be_ref, w_ref,
             out_ref):
        inv = inv_ref[:N]
        planes = []
        for tt in range(P1):
            s = part_ref[0, tt, :N, :]
            for gg in range(1, NG1):
                s = s + part_ref[gg, tt, :N, :]
            s = s + inv * ht_ref[tt, :N, :] + b_ref[0, tt * 8:(tt + 1) * 8]
            planes.append(s)
        h = jnp.concatenate(planes, axis=1)
        h = jnp.maximum(_bn(h, g_ref[0], be_ref[0]), 0.0)
        h2 = jnp.dot(h, w_ref[...], preferred_element_type=jnp.float32)
        for tt in range(P2):
            out_ref[tt, :N] = h2[:, tt * 8:(tt + 1) * 8]
            out_ref[tt, N:] = jnp.zeros((NPAD - N, 8), jnp.float32)

    return pl.pallas_call(
        body,
        out_shape=jax.ShapeDtypeStruct((F2 // 8, NPAD, 8), jnp.float32))


@functools.lru_cache(maxsize=None)
def _make_tc_post(N, NPAD, F2, FD1, DOUT):
    P2, NG2 = F2 // 8, NW // (F2 // 8)

    def body(part_ref, ht_ref, inv_ref, b2_ref, g2_ref, be2_ref,
             wd1_ref, bd1_ref, gd1_ref, bed1_ref, wd2_ref, bd2_ref,
             z_ref, recon_ref):
        inv = inv_ref[:N]
        planes = []
        for tt in range(P2):
            s = part_ref[0, tt, :N, :]
            for gg in range(1, NG2):
                s = s + part_ref[gg, tt, :N, :]
            s = s + inv * ht_ref[tt, :N, :] + b2_ref[0, tt * 8:(tt + 1) * 8]
            planes.append(s)
        h = jnp.concatenate(planes, axis=1)
        z = jnp.maximum(_bn(h, g2_ref[0], be2_ref[0]), 0.0)
        z_ref[...] = z
        d = jnp.dot(z, wd1_ref[...],
                    preferred_element_type=jnp.float32) + bd1_ref[0]
        d = jnp.maximum(_bn(d, gd1_ref[0], bed1_ref[0]), 0.0)
        recon_ref[...] = jnp.dot(d, wd2_ref[...],
                                 preferred_element_type=jnp.float32) + bd2_ref[0]

    return pl.pallas_call(
        body,
        out_shape=[jax.ShapeDtypeStruct((N, F2), jnp.float32),
                   jax.ShapeDtypeStruct((N, DOUT), jnp.float32)])


def kernel(x, edge_index, edge_attr, W1, b1, g1, be1, W2, b2, g2, be2,
           Wd1, bd1, gd1, bed1, Wd2, bd2):
    N, DIN = x.shape
    E = edge_attr.shape[0]
    F1, F2 = W1.shape[1], W2.shape[1]
    FD1, DOUT = Wd1.shape[1], Wd2.shape[1]
    NPAD = ((N + 127) // 128) * 128

    src = edge_index[0]
    dst = edge_index[1]

    norm, invp = _make_sc_norm(E, NPAD)(src, dst, edge_attr)
    inv2 = invp.reshape(NPAD, 1)

    h1t = _make_tc_pre(N, NPAD, DIN, F1)(x, W1)
    part1 = _make_sc_agg(E, NPAD, F1 // 8)(
        h1t.reshape(F1 // 8 * NPAD, 8), src, dst, norm)
    part1 = part1.reshape(NW // (F1 // 8), F1 // 8, NPAD, 8)

    h2t = _make_tc_mid(N, NPAD, F1, F2)(
        part1, h1t, inv2, b1.reshape(1, F1), g1.reshape(1, F1),
        be1.reshape(1, F1), W2)
    part2 = _make_sc_agg(E, NPAD, F2 // 8)(
        h2t.reshape(F2 // 8 * NPAD, 8), src, dst, norm)
    part2 = part2.reshape(NW // (F2 // 8), F2 // 8, NPAD, 8)

    z, recon = _make_tc_post(N, NPAD, F2, FD1, DOUT)(
        part2, h2t, inv2, b2.reshape(1, F2), g2.reshape(1, F2),
        be2.reshape(1, F2), Wd1, bd1.reshape(1, FD1), gd1.reshape(1, FD1),
        bed1.reshape(1, FD1), Wd2, bd2.reshape(1, DOUT))
    return (recon, z)


# speedup vs baseline: 7.6741x; 7.6741x over previous
import functools

import jax
import jax.numpy as jnp
from jax import lax
from jax.experimental import pallas as pl
from jax.experimental.pallas import tpu as pltpu
from jax.experimental.pallas import tpu_sc as plsc

NLANE = 16
NTILE = 16
NCORE = 2
NW = NCORE * NTILE


def _mesh():
    return plsc.VectorSubcoreMesh(core_axis_name="c", subcore_axis_name="s")


_SC_PARAMS = dict(needs_layout_passes=False, use_tc_tiling_on_sc=False)


@functools.lru_cache(maxsize=None)
def _make_sc_norm(E, NPAD):
    SLICE = NPAD // NTILE
    ED = E // NTILE
    ECH = 2000
    EN = E // NW
    NCH = 2000
    assert ED % ECH == 0 and EN % NCH == 0 and SLICE % NLANE == 0

    def body(src_hbm, dst_hbm, ew_hbm, norm_hbm, inv_hbm,
             acc, dis_loc, sb, db, wb, nb, psum, dslice, islice,
             sp_part, sp_dis):
        cid = lax.axis_index("c")
        sid = lax.axis_index("s")
        iota = lax.iota(jnp.int32, NLANE)
        plane = jnp.bitwise_and(iota, 7)
        mlow = iota < 8
        mhigh = jnp.logical_not(mlow)
        fzero = jnp.zeros((NLANE,), jnp.float32)

        def zloop(i, c):
            acc[pl.ds(i * NLANE, NLANE)] = fzero
            return c
        lax.fori_loop(0, (8 * NPAD) // NLANE, zloop, 0)

        def deg_chunk(k, c):
            base = sid * ED + k * ECH
            pltpu.sync_copy(dst_hbm.at[pl.ds(base, ECH)], db)
            pltpu.sync_copy(ew_hbm.at[pl.ds(base, ECH)], wb)

            def deg16(i, cc):
                d16 = db[pl.ds(i * NLANE, NLANE)]
                w16 = wb[pl.ds(i * NLANE, NLANE)]
                fidx = plane * NPAD + d16
                plsc.addupdate_scatter(acc, [fidx], w16, mask=mlow)
                plsc.addupdate_scatter(acc, [fidx], w16, mask=mhigh)
                return cc
            lax.fori_loop(0, ECH // NLANE, deg16, 0)
            return c
        lax.fori_loop(0, ED // ECH, deg_chunk, 0)

        def red(i, c):
            s = acc[pl.ds(i * NLANE, NLANE)]
            for p in range(1, 8):
                s = s + acc[pl.ds(p * NPAD + i * NLANE, NLANE)]
            dis_loc[pl.ds(i * NLANE, NLANE)] = s
            return c
        lax.fori_loop(0, NPAD // NLANE, red, 0)

        pltpu.sync_copy(dis_loc, sp_part.at[pl.ds(sid * NPAD, NPAD)])
        plsc.subcore_barrier()

        for r in range(NTILE):
            pltpu.sync_copy(sp_part.at[pl.ds(r * NPAD + sid * SLICE, SLICE)],
                            psum.at[pl.ds(r * SLICE, SLICE)])

        magic = jnp.full((NLANE,), 0x5F3759DF, jnp.int32)

        def disloop(i, c):
            s = psum[pl.ds(i * NLANE, NLANE)]
            for r in range(1, NTILE):
                s = s + psum[pl.ds(r * SLICE + i * NLANE, NLANE)]
            d = s + 1.0
            ibits = plsc.bitcast(d, jnp.int32)
            y = plsc.bitcast(magic - lax.shift_right_logical(ibits, 1),
                             jnp.float32)
            for _ in range(3):
                y = y * (1.5 - 0.5 * d * y * y)
            dslice[pl.ds(i * NLANE, NLANE)] = y
            islice[pl.ds(i * NLANE, NLANE)] = y * y
            return c
        lax.fori_loop(0, SLICE // NLANE, disloop, 0)

        pltpu.sync_copy(dslice, sp_dis.at[pl.ds(sid * SLICE, SLICE)])

        @pl.when(cid == 0)
        def _():
            pltpu.sync_copy(islice, inv_hbm.at[pl.ds(sid * SLICE, SLICE)])

        plsc.subcore_barrier()
        pltpu.sync_copy(sp_dis, dis_loc)

        wid = cid * NTILE + sid

        def nchunk(k, c):
            base = wid * EN + k * NCH
            pltpu.sync_copy(src_hbm.at[pl.ds(base, NCH)], sb)
            pltpu.sync_copy(dst_hbm.at[pl.ds(base, NCH)], db)
            pltpu.sync_copy(ew_hbm.at[pl.ds(base, NCH)], wb)

            def n16(i, cc):
                s16 = sb[pl.ds(i * NLANE, NLANE)]
                d16 = db[pl.ds(i * NLANE, NLANE)]
                w16 = wb[pl.ds(i * NLANE, NLANE)]
                a = plsc.load_gather(dis_loc, [s16])
                b = plsc.load_gather(dis_loc, [d16])
                nb[pl.ds(i * NLANE, NLANE)] = a * w16 * b
                return cc
            lax.fori_loop(0, NCH // NLANE, n16, 0)
            pltpu.sync_copy(nb, norm_hbm.at[pl.ds(base, NCH)])
            return c
        lax.fori_loop(0, EN // NCH, nchunk, 0)

    return pl.kernel(
        body,
        out_type=[jax.ShapeDtypeStruct((E,), jnp.float32),
                  jax.ShapeDtypeStruct((NPAD,), jnp.float32)],
        mesh=_mesh(),
        compiler_params=pltpu.CompilerParams(**_SC_PARAMS),
        scratch_types=[
            pltpu.VMEM((8 * NPAD,), jnp.float32),
            pltpu.VMEM((NPAD,), jnp.float32),
            pltpu.VMEM((NCH,), jnp.int32),
            pltpu.VMEM((NCH,), jnp.int32),
            pltpu.VMEM((NCH,), jnp.float32),
            pltpu.VMEM((NCH,), jnp.float32),
            pltpu.VMEM((NTILE * SLICE,), jnp.float32),
            pltpu.VMEM((SLICE,), jnp.float32),
            pltpu.VMEM((SLICE,), jnp.float32),
            pltpu.VMEM_SHARED((NTILE * NPAD,), jnp.float32),
            pltpu.VMEM_SHARED((NPAD,), jnp.float32),
        ],
    )


@functools.lru_cache(maxsize=None)
def _make_sc_agg(E, NPAD, F):
    P = F // 8
    GPS = NTILE // P
    NG = NCORE * GPS
    EG = E // NG
    CH = 2000
    SLEN = 80
    NS = CH // SLEN
    NC = EG // CH
    NROWS = NPAD // GPS
    RB = 1280
    assert EG % CH == 0 and CH % SLEN == 0 and NROWS % RB == 0

    def body(tab_hbm, src_hbm, dst_hbm, norm_hbm, out_hbm,
             acc, rows, rbuf, sb, db, nbuf, sp_stage, sem):
        cid = lax.axis_index("c")
        sid = lax.axis_index("s")
        gl = sid // P
        t = sid - gl * P
        g = cid * GPS + gl
        iota = lax.iota(jnp.int32, NLANE)
        col = jnp.bitwise_and(iota, 7)
        c01 = lax.shift_right_logical(iota, 3)
        mlow = iota < 8
        mhigh = jnp.logical_not(mlow)
        fzero = jnp.zeros((NLANE,), jnp.float32)
        two = jnp.full((NLANE,), 2, jnp.int32)

        def zloop(i, c):
            plsc.store_scatter(acc, [two * i + c01, col], fzero)
            return c
        lax.fori_loop(0, NPAD // 2, zloop, 0)

        toff = jnp.full((NLANE,), 1, jnp.int32) * t

        def chunk(k, c):
            base = g * EG + k * CH
            pltpu.sync_copy(src_hbm.at[pl.ds(base, CH)], sb)
            pltpu.sync_copy(dst_hbm.at[pl.ds(base, CH)], db)
            pltpu.sync_copy(norm_hbm.at[pl.ds(base, CH)], nbuf)

            def adj(i, cc):
                s16 = sb[pl.ds(i * NLANE, NLANE)]
                sb[pl.ds(i * NLANE, NLANE)] = s16 * P + toff
                return cc
            lax.fori_loop(0, CH // NLANE, adj, 0)

            for j in range(NS):
                pltpu.async_copy(
                    tab_hbm.at[sb.at[pl.ds(j * SLEN, SLEN)]],
                    rows.at[pl.ds(j * SLEN, SLEN)], sem)
            for j in range(NS):
                pltpu.make_async_copy(
                    tab_hbm.at[sb.at[pl.ds(j * SLEN, SLEN)]],
                    rows.at[pl.ds(j * SLEN, SLEN)], sem).wait()

            def pair(p, cc):
                psel = two * p + c01
                dstp = plsc.load_gather(db, [psel])
                normp = plsc.load_gather(nbuf, [psel])
                r16 = plsc.load_gather(rows, [psel, col])
                val = r16 * normp
                plsc.addupdate_scatter(acc, [dstp, col], val, mask=mlow)
                plsc.addupdate_scatter(acc, [dstp, col], val, mask=mhigh)
                return cc
            lax.fori_loop(0, CH // 2, pair, 0)
            return c
        lax.fori_loop(0, NC, chunk, 0)

        qbase = gl * NROWS
        for o in range(1, GPS):
            src_tile = ((gl + o) % GPS) * P + t
            give = ((gl + GPS - o) % GPS) * NROWS

            def fold_blk(b, c):
                pltpu.sync_copy(acc.at[pl.ds(give + b * RB, RB)],
                                sp_stage.at[sid])
                plsc.subcore_barrier()
                pltpu.sync_copy(sp_stage.at[src_tile], rbuf)
                rowb = qbase + b * RB

                def fold16(i, cc):
                    rp = two * i + c01
                    v = plsc.load_gather(rbuf, [rp, col])
                    tp = rp + rowb
                    plsc.addupdate_scatter(acc, [tp, col], v, mask=mlow)
                    plsc.addupdate_scatter(acc, [tp, col], v, mask=mhigh)
                    return cc
                lax.fori_loop(0, RB // 2, fold16, 0)
                plsc.subcore_barrier()
                return c
            lax.fori_loop(0, NROWS // RB, fold_blk, 0)

        pltpu.sync_copy(acc.at[pl.ds(qbase, NROWS)],
                        out_hbm.at[cid, pl.ds(qbase, NROWS),
                                   pl.ds(t * 8, 8)])

    return pl.kernel(
        body,
        out_type=jax.ShapeDtypeStruct((NCORE, NPAD, F), jnp.float32),
        mesh=_mesh(),
        compiler_params=pltpu.CompilerParams(**_SC_PARAMS),
        scratch_types=[
            pltpu.VMEM((NPAD, 8), jnp.float32),
            pltpu.VMEM((CH, 8), jnp.float32),
            pltpu.VMEM((RB, 8), jnp.float32),
            pltpu.VMEM((CH,), jnp.int32),
            pltpu.VMEM((CH,), jnp.int32),
            pltpu.VMEM((CH,), jnp.float32),
            pltpu.VMEM_SHARED((NTILE, RB, 8), jnp.float32),
            pltpu.SemaphoreType.DMA,
        ],
    )


@functools.lru_cache(maxsize=None)
def _make_tc_pre(N, NPAD, DIN, F):
    def body(x_ref, w_ref, out_ref):
        h = jnp.dot(x_ref[...], w_ref[...],
                    preferred_element_type=jnp.float32)
        out_ref[:N] = h
        out_ref[N:] = jnp.zeros((NPAD - N, F), jnp.float32)

    return pl.pallas_call(
        body,
        out_shape=jax.ShapeDtypeStruct((NPAD, F), jnp.float32))


def _bn(h, g, b):
    m = jnp.mean(h, axis=0, keepdims=True)
    v = jnp.mean((h - m) * (h - m), axis=0, keepdims=True)
    return (h - m) * lax.rsqrt(v + 1e-5) * g + b


@functools.lru_cache(maxsize=None)
def _make_tc_mid(N, NPAD, F1, F2):
    def body(part_ref, ht_ref, inv_ref, b_ref, g_ref, be_ref, w_ref,
             out_ref):
        inv = inv_ref[:N]
        h = (part_ref[0, :N, :] + part_ref[1, :N, :]
             + inv * ht_ref[:N, :] + b_ref[0])
        h = jnp.maximum(_bn(h, g_ref[0], be_ref[0]), 0.0)
        h2 = jnp.dot(h, w_ref[...], preferred_element_type=jnp.float32)
        out_ref[:N] = h2
        out_ref[N:] = jnp.zeros((NPAD - N, F2), jnp.float32)

    return pl.pallas_call(
        body,
        out_shape=jax.ShapeDtypeStruct((NPAD, F2), jnp.float32))


@functools.lru_cache(maxsize=None)
def _make_tc_post(N, NPAD, F2, FD1, DOUT):
    def body(part_ref, ht_ref, inv_ref, b2_ref, g2_ref, be2_ref,
             wd1_ref, bd1_ref, gd1_ref, bed1_ref, wd2_ref, bd2_ref,
             z_ref, recon_ref):
        inv = inv_ref[:N]
        h = (part_ref[0, :N, :] + part_ref[1, :N, :]
             + inv * ht_ref[:N, :] + b2_ref[0])
        z = jnp.maximum(_bn(h, g2_ref[0], be2_ref[0]), 0.0)
        z_ref[...] = z
        d = jnp.dot(z, wd1_ref[...],
                    preferred_element_type=jnp.float32) + bd1_ref[0]
        d = jnp.maximum(_bn(d, gd1_ref[0], bed1_ref[0]), 0.0)
        recon_ref[...] = jnp.dot(d, wd2_ref[...],
                                 preferred_element_type=jnp.float32) + bd2_ref[0]

    return pl.pallas_call(
        body,
        out_shape=[jax.ShapeDtypeStruct((N, F2), jnp.float32),
                   jax.ShapeDtypeStruct((N, DOUT), jnp.float32)])


def kernel(x, edge_index, edge_attr, W1, b1, g1, be1, W2, b2, g2, be2,
           Wd1, bd1, gd1, bed1, Wd2, bd2):
    N, DIN = x.shape
    E = edge_attr.shape[0]
    F1, F2 = W1.shape[1], W2.shape[1]
    FD1, DOUT = Wd1.shape[1], Wd2.shape[1]
    NPAD = ((N + 255) // 256) * 256

    src = edge_index[0]
    dst = edge_index[1]

    norm, invp = _make_sc_norm(E, NPAD)(src, dst, edge_attr)
    inv2 = invp.reshape(NPAD, 1)

    h1 = _make_tc_pre(N, NPAD, DIN, F1)(x, W1)
    part1 = _make_sc_agg(E, NPAD, F1)(
        h1.reshape(NPAD * (F1 // 8), 8), src, dst, norm)

    h2 = _make_tc_mid(N, NPAD, F1, F2)(
        part1, h1, inv2, b1.reshape(1, F1), g1.reshape(1, F1),
        be1.reshape(1, F1), W2)
    part2 = _make_sc_agg(E, NPAD, F2)(
        h2.reshape(NPAD * (F2 // 8), 8), src, dst, norm)

    z, recon = _make_tc_post(N, NPAD, F2, FD1, DOUT)(
        part2, h2, inv2, b2.reshape(1, F2), g2.reshape(1, F2),
        be2.reshape(1, F2), Wd1, bd1.reshape(1, FD1), gd1.reshape(1, FD1),
        bed1.reshape(1, FD1), Wd2, bd2.reshape(1, DOUT))
    return (recon, z)


# speedup vs baseline: 10.4010x; 1.3553x over previous
import functools

import jax
import jax.numpy as jnp
from jax import lax
from jax.experimental import pallas as pl
from jax.experimental.pallas import tpu as pltpu
from jax.experimental.pallas import tpu_sc as plsc

NLANE = 16
NTILE = 16
NCORE = 2
NW = NCORE * NTILE


def _mesh():
    return plsc.VectorSubcoreMesh(core_axis_name="c", subcore_axis_name="s")


_SC_PARAMS = dict(needs_layout_passes=False, use_tc_tiling_on_sc=False)


@functools.lru_cache(maxsize=None)
def _make_sc_norm(E, NPAD):
    SLICE = NPAD // NTILE
    ED = E // NTILE
    ECH = 2000
    EN = E // NW
    NCH = 2000
    assert ED % ECH == 0 and EN % NCH == 0 and SLICE % NLANE == 0

    def body(src_hbm, dst_hbm, ew_hbm, norm_hbm, inv_hbm,
             acc, dis_loc, sb, db, wb, nb, psum, dslice, islice,
             sp_part, sp_dis):
        cid = lax.axis_index("c")
        sid = lax.axis_index("s")
        iota = lax.iota(jnp.int32, NLANE)
        plane = jnp.bitwise_and(iota, 7)
        mlow = iota < 8
        mhigh = jnp.logical_not(mlow)
        fzero = jnp.zeros((NLANE,), jnp.float32)

        def zloop(i, c):
            for u in range(8):
                acc[pl.ds((8 * i + u) * NLANE, NLANE)] = fzero
            return c
        lax.fori_loop(0, (8 * NPAD) // (8 * NLANE), zloop, 0)

        def deg_chunk(k, c):
            base = sid * ED + k * ECH
            pltpu.sync_copy(dst_hbm.at[pl.ds(base, ECH)], db)
            pltpu.sync_copy(ew_hbm.at[pl.ds(base, ECH)], wb)

            def deg16(i, cc):
                for u in range(5):
                    sl = pl.ds((5 * i + u) * NLANE, NLANE)
                    d16 = db[sl]
                    w16 = wb[sl]
                    fidx = plane * NPAD + d16
                    plsc.addupdate_scatter(acc, [fidx], w16, mask=mlow)
                    plsc.addupdate_scatter(acc, [fidx], w16, mask=mhigh)
                return cc
            lax.fori_loop(0, ECH // (5 * NLANE), deg16, 0)
            return c
        lax.fori_loop(0, ED // ECH, deg_chunk, 0)

        def red(i, c):
            for u in range(4):
                j = 4 * i + u
                s = acc[pl.ds(j * NLANE, NLANE)]
                for p in range(1, 8):
                    s = s + acc[pl.ds(p * NPAD + j * NLANE, NLANE)]
                dis_loc[pl.ds(j * NLANE, NLANE)] = s
            return c
        lax.fori_loop(0, NPAD // (4 * NLANE), red, 0)

        pltpu.sync_copy(dis_loc, sp_part.at[pl.ds(sid * NPAD, NPAD)])
        plsc.subcore_barrier()

        for r in range(NTILE):
            pltpu.sync_copy(sp_part.at[pl.ds(r * NPAD + sid * SLICE, SLICE)],
                            psum.at[pl.ds(r * SLICE, SLICE)])

        magic = jnp.full((NLANE,), 0x5F3759DF, jnp.int32)

        def disloop(i, c):
            s = psum[pl.ds(i * NLANE, NLANE)]
            for r in range(1, NTILE):
                s = s + psum[pl.ds(r * SLICE + i * NLANE, NLANE)]
            d = s + 1.0
            ibits = plsc.bitcast(d, jnp.int32)
            y = plsc.bitcast(magic - lax.shift_right_logical(ibits, 1),
                             jnp.float32)
            for _ in range(3):
                y = y * (1.5 - 0.5 * d * y * y)
            dslice[pl.ds(i * NLANE, NLANE)] = y
            islice[pl.ds(i * NLANE, NLANE)] = y * y
            return c
        lax.fori_loop(0, SLICE // NLANE, disloop, 0)

        pltpu.sync_copy(dslice, sp_dis.at[pl.ds(sid * SLICE, SLICE)])

        @pl.when(cid == 0)
        def _():
            pltpu.sync_copy(islice, inv_hbm.at[pl.ds(sid * SLICE, SLICE)])

        plsc.subcore_barrier()
        pltpu.sync_copy(sp_dis, dis_loc)

        wid = cid * NTILE + sid

        def nchunk(k, c):
            base = wid * EN + k * NCH
            pltpu.sync_copy(src_hbm.at[pl.ds(base, NCH)], sb)
            pltpu.sync_copy(dst_hbm.at[pl.ds(base, NCH)], db)
            pltpu.sync_copy(ew_hbm.at[pl.ds(base, NCH)], wb)

            def n16(i, cc):
                for u in range(5):
                    sl = pl.ds((5 * i + u) * NLANE, NLANE)
                    a = plsc.load_gather(dis_loc, [sb[sl]])
                    b = plsc.load_gather(dis_loc, [db[sl]])
                    nb[sl] = a * wb[sl] * b
                return cc
            lax.fori_loop(0, NCH // (5 * NLANE), n16, 0)
            pltpu.sync_copy(nb, norm_hbm.at[pl.ds(base, NCH)])
            return c
        lax.fori_loop(0, EN // NCH, nchunk, 0)

    return pl.kernel(
        body,
        out_type=[jax.ShapeDtypeStruct((E,), jnp.float32),
                  jax.ShapeDtypeStruct((NPAD,), jnp.float32)],
        mesh=_mesh(),
        compiler_params=pltpu.CompilerParams(**_SC_PARAMS),
        scratch_types=[
            pltpu.VMEM((8 * NPAD,), jnp.float32),
            pltpu.VMEM((NPAD,), jnp.float32),
            pltpu.VMEM((NCH,), jnp.int32),
            pltpu.VMEM((NCH,), jnp.int32),
            pltpu.VMEM((NCH,), jnp.float32),
            pltpu.VMEM((NCH,), jnp.float32),
            pltpu.VMEM((NTILE * SLICE,), jnp.float32),
            pltpu.VMEM((SLICE,), jnp.float32),
            pltpu.VMEM((SLICE,), jnp.float32),
            pltpu.VMEM_SHARED((NTILE * NPAD,), jnp.float32),
            pltpu.VMEM_SHARED((NPAD,), jnp.float32),
        ],
    )


@functools.lru_cache(maxsize=None)
def _make_sc_agg(E, NPAD, F):
    P = F // 8
    GPS = NTILE // P
    NG = NCORE * GPS
    EG = E // NG
    NC = 50
    CH = EG // NC
    SLEN = 80
    NS = CH // SLEN
    NROWS = NPAD // GPS
    RB = 640
    UN = 8
    assert EG % NC == 0 and CH % SLEN == 0 and NROWS % RB == 0
    assert NC % 2 == 0 and (CH // 2) % UN == 0 and CH % (2 * NLANE) == 0

    def body(tab_hbm, src_hbm, dst_hbm, norm_hbm, out_hbm,
             acc, rows0, rows1, rbuf, sb0, sb1, db0, db1, nb0, nb1,
             sp_stage, seml0, seml1, semr0, semr1):
        rows = (rows0, rows1)
        sb = (sb0, sb1)
        db = (db0, db1)
        nb = (nb0, nb1)
        seml = (seml0, seml1)
        semr = (semr0, semr1)
        cid = lax.axis_index("c")
        sid = lax.axis_index("s")
        gl = sid // P
        t = sid - gl * P
        g = cid * GPS + gl
        iota = lax.iota(jnp.int32, NLANE)
        col = jnp.bitwise_and(iota, 7)
        c01 = lax.shift_right_logical(iota, 3)
        c01u = [c01 + 2 * u for u in range(UN)]
        mlow = iota < 8
        mhigh = jnp.logical_not(mlow)
        fzero = jnp.zeros((NLANE,), jnp.float32)
        two = jnp.full((NLANE,), 2, jnp.int32)

        def zloop(i, c):
            for u in range(8):
                plsc.store_scatter(acc, [two * (8 * i + u) + c01, col],
                                   fzero)
            return c
        lax.fori_loop(0, NPAD // 16, zloop, 0)

        toff = jnp.full((NLANE,), 1, jnp.int32) * t

        def lin_issue(k, s):
            base = g * EG + k * CH
            pltpu.async_copy(src_hbm.at[pl.ds(base, CH)], sb[s], seml[s])
            pltpu.async_copy(dst_hbm.at[pl.ds(base, CH)], db[s], seml[s])
            pltpu.async_copy(norm_hbm.at[pl.ds(base, CH)], nb[s], seml[s])

        def lin_wait(s):
            pltpu.make_async_copy(src_hbm.at[pl.ds(0, CH)], sb[s],
                                  seml[s]).wait()
            pltpu.make_async_copy(dst_hbm.at[pl.ds(0, CH)], db[s],
                                  seml[s]).wait()
            pltpu.make_async_copy(norm_hbm.at[pl.ds(0, CH)], nb[s],
                                  seml[s]).wait()

        def adj(s):
            def adj1(i, cc):
                for u in range(2):
                    sl = pl.ds((2 * i + u) * NLANE, NLANE)
                    sb[s][sl] = sb[s][sl] * P + toff
                return cc
            lax.fori_loop(0, CH // (2 * NLANE), adj1, 0)

        def rows_fire(s):
            for j in range(NS):
                pltpu.async_copy(
                    tab_hbm.at[sb[s].at[pl.ds(j * SLEN, SLEN)]],
                    rows[s].at[pl.ds(j * SLEN, SLEN)], semr[s])

        def rows_wait(s):
            for j in range(NS):
                pltpu.make_async_copy(
                    tab_hbm.at[sb[s].at[pl.ds(j * SLEN, SLEN)]],
                    rows[s].at[pl.ds(j * SLEN, SLEN)], semr[s]).wait()

        def compute(s):
            def pairs(q, cc):
                b16 = jnp.full((NLANE,), 2 * UN, jnp.int32) * q
                for u in range(UN):
                    psel = b16 + c01u[u]
                    dstp = plsc.load_gather(db[s], [psel])
                    normp = plsc.load_gather(nb[s], [psel])
                    r16 = plsc.load_gather(rows[s], [psel, col])
                    val = r16 * normp
                    plsc.addupdate_scatter(acc, [dstp, col], val,
                                           mask=mlow)
                    plsc.addupdate_scatter(acc, [dstp, col], val,
                                           mask=mhigh)
                return cc
            lax.fori_loop(0, CH // (2 * UN), pairs, 0)

        lin_issue(0, 0)
        lin_wait(0)
        adj(0)
        rows_fire(0)
        lin_issue(1, 1)

        def piter(ko, c):
            for u2 in range(2):
                k = ko * 2 + u2
                s = u2
                rows_wait(s)

                @pl.when(k + 1 < NC)
                def _():
                    lin_wait(1 - s)
                    adj(1 - s)
                    rows_fire(1 - s)

                compute(s)

                @pl.when(k + 2 < NC)
                def _():
                    lin_issue(k + 2, s)
            return c
        lax.fori_loop(0, NC // 2, piter, 0)

        qbase = gl * NROWS
        for o in range(1, GPS):
            src_tile = ((gl + o) % GPS) * P + t
            give = ((gl + GPS - o) % GPS) * NROWS

            def fold_blk(b, c):
                pltpu.sync_copy(acc.at[pl.ds(give + b * RB, RB)],
                                sp_stage.at[sid])
                plsc.subcore_barrier()
                pltpu.sync_copy(sp_stage.at[src_tile], rbuf)
                rowb = qbase + b * RB

                def fold16(i, cc):
                    for u in range(8):
                        rp = two * (8 * i + u) + c01
                        v = plsc.load_gather(rbuf, [rp, col])
                        tp = rp + rowb
                        plsc.addupdate_scatter(acc, [tp, col], v,
                                               mask=mlow)
                        plsc.addupdate_scatter(acc, [tp, col], v,
                                               mask=mhigh)
                    return cc
                lax.fori_loop(0, RB // 16, fold16, 0)
                plsc.subcore_barrier()
                return c
            lax.fori_loop(0, NROWS // RB, fold_blk, 0)

        pltpu.sync_copy(acc.at[pl.ds(qbase, NROWS)],
                        out_hbm.at[cid, pl.ds(qbase, NROWS),
                                   pl.ds(t * 8, 8)])

    return pl.kernel(
        body,
        out_type=jax.ShapeDtypeStruct((NCORE, NPAD, F), jnp.float32),
        mesh=_mesh(),
        compiler_params=pltpu.CompilerParams(**_SC_PARAMS),
        scratch_types=[
            pltpu.VMEM((NPAD, 8), jnp.float32),
            pltpu.VMEM((CH, 8), jnp.float32),
            pltpu.VMEM((CH, 8), jnp.float32),
            pltpu.VMEM((RB, 8), jnp.float32),
            pltpu.VMEM((CH,), jnp.int32),
            pltpu.VMEM((CH,), jnp.int32),
            pltpu.VMEM((CH,), jnp.int32),
            pltpu.VMEM((CH,), jnp.int32),
            pltpu.VMEM((CH,), jnp.float32),
            pltpu.VMEM((CH,), jnp.float32),
            pltpu.VMEM_SHARED((NTILE, RB, 8), jnp.float32),
            pltpu.SemaphoreType.DMA,
            pltpu.SemaphoreType.DMA,
            pltpu.SemaphoreType.DMA,
            pltpu.SemaphoreType.DMA,
        ],
    )


@functools.lru_cache(maxsize=None)
def _make_tc_pre(N, NPAD, DIN, F):
    def body(x_ref, w_ref, out_ref):
        h = jnp.dot(x_ref[...], w_ref[...],
                    preferred_element_type=jnp.float32)
        out_ref[:N] = h
        out_ref[N:] = jnp.zeros((NPAD - N, F), jnp.float32)

    return pl.pallas_call(
        body,
        out_shape=jax.ShapeDtypeStruct((NPAD, F), jnp.float32))


def _bn(h, g, b):
    m = jnp.mean(h, axis=0, keepdims=True)
    v = jnp.mean((h - m) * (h - m), axis=0, keepdims=True)
    return (h - m) * lax.rsqrt(v + 1e-5) * g + b


@functools.lru_cache(maxsize=None)
def _make_tc_mid(N, NPAD, F1, F2):
    def body(part_ref, ht_ref, inv_ref, b_ref, g_ref, be_ref, w_ref,
             out_ref):
        inv = inv_ref[:N]
        h = (part_ref[0, :N, :] + part_ref[1, :N, :]
             + inv * ht_ref[:N, :] + b_ref[0])
        h = jnp.maximum(_bn(h, g_ref[0], be_ref[0]), 0.0)
        h2 = jnp.dot(h, w_ref[...], preferred_element_type=jnp.float32)
        out_ref[:N] = h2
        out_ref[N:] = jnp.zeros((NPAD - N, F2), jnp.float32)

    return pl.pallas_call(
        body,
        out_shape=jax.ShapeDtypeStruct((NPAD, F2), jnp.float32))


@functools.lru_cache(maxsize=None)
def _make_tc_post(N, NPAD, F2, FD1, DOUT):
    def body(part_ref, ht_ref, inv_ref, b2_ref, g2_ref, be2_ref,
             wd1_ref, bd1_ref, gd1_ref, bed1_ref, wd2_ref, bd2_ref,
             z_ref, recon_ref):
        inv = inv_ref[:N]
        h = (part_ref[0, :N, :] + part_ref[1, :N, :]
             + inv * ht_ref[:N, :] + b2_ref[0])
        z = jnp.maximum(_bn(h, g2_ref[0], be2_ref[0]), 0.0)
        z_ref[...] = z
        d = jnp.dot(z, wd1_ref[...],
                    preferred_element_type=jnp.float32) + bd1_ref[0]
        d = jnp.maximum(_bn(d, gd1_ref[0], bed1_ref[0]), 0.0)
        recon_ref[...] = jnp.dot(d, wd2_ref[...],
                                 preferred_element_type=jnp.float32) + bd2_ref[0]

    return pl.pallas_call(
        body,
        out_shape=[jax.ShapeDtypeStruct((N, F2), jnp.float32),
                   jax.ShapeDtypeStruct((N, DOUT), jnp.float32)])


def kernel(x, edge_index, edge_attr, W1, b1, g1, be1, W2, b2, g2, be2,
           Wd1, bd1, gd1, bed1, Wd2, bd2):
    N, DIN = x.shape
    E = edge_attr.shape[0]
    F1, F2 = W1.shape[1], W2.shape[1]
    FD1, DOUT = Wd1.shape[1], Wd2.shape[1]
    NPAD = ((N + 255) // 256) * 256

    src = edge_index[0]
    dst = edge_index[1]

    norm, invp = _make_sc_norm(E, NPAD)(src, dst, edge_attr)
    inv2 = invp.reshape(NPAD, 1)

    h1 = _make_tc_pre(N, NPAD, DIN, F1)(x, W1)
    part1 = _make_sc_agg(E, NPAD, F1)(
        h1.reshape(NPAD * (F1 // 8), 8), src, dst, norm)

    h2 = _make_tc_mid(N, NPAD, F1, F2)(
        part1, h1, inv2, b1.reshape(1, F1), g1.reshape(1, F1),
        be1.reshape(1, F1), W2)
    part2 = _make_sc_agg(E, NPAD, F2)(
        h2.reshape(NPAD * (F2 // 8), 8), src, dst, norm)

    z, recon = _make_tc_post(N, NPAD, F2, FD1, DOUT)(
        part2, h2, inv2, b2.reshape(1, F2), g2.reshape(1, F2),
        be2.reshape(1, F2), Wd1, bd1.reshape(1, FD1), gd1.reshape(1, FD1),
        bed1.reshape(1, FD1), Wd2, bd2.reshape(1, DOUT))
    return (recon, z)


# speedup vs baseline: 10.4347x; 1.0032x over previous
import functools

import jax
import jax.numpy as jnp
from jax import lax
from jax.experimental import pallas as pl
from jax.experimental.pallas import tpu as pltpu
from jax.experimental.pallas import tpu_sc as plsc

NLANE = 16
NTILE = 16
NCORE = 2
NW = NCORE * NTILE


def _mesh():
    return plsc.VectorSubcoreMesh(core_axis_name="c", subcore_axis_name="s")


_SC_PARAMS = dict(needs_layout_passes=False, use_tc_tiling_on_sc=False)


@functools.lru_cache(maxsize=None)
def _make_sc_norm(E, NPAD):
    SLICE = NPAD // NTILE
    ED = E // NTILE
    ECH = 2000
    EN = E // NW
    NCH = 2000
    assert ED % ECH == 0 and EN % NCH == 0 and SLICE % NLANE == 0

    def body(src_hbm, dst_hbm, ew_hbm, norm_hbm, inv_hbm,
             acc, dis_loc, sb, db, wb, nb, psum, dslice, islice,
             sp_part, sp_dis):
        cid = lax.axis_index("c")
        sid = lax.axis_index("s")
        iota = lax.iota(jnp.int32, NLANE)
        plane = jnp.bitwise_and(iota, 7)
        mlow = iota < 8
        mhigh = jnp.logical_not(mlow)
        fzero = jnp.zeros((NLANE,), jnp.float32)

        def zloop(i, c):
            for u in range(8):
                acc[pl.ds((8 * i + u) * NLANE, NLANE)] = fzero
            return c
        lax.fori_loop(0, (8 * NPAD) // (8 * NLANE), zloop, 0)

        def deg_chunk(k, c):
            base = sid * ED + k * ECH
            pltpu.sync_copy(dst_hbm.at[pl.ds(base, ECH)], db)
            pltpu.sync_copy(ew_hbm.at[pl.ds(base, ECH)], wb)

            def deg16(i, cc):
                for u in range(5):
                    sl = pl.ds((5 * i + u) * NLANE, NLANE)
                    d16 = db[sl]
                    w16 = wb[sl]
                    fidx = plane * NPAD + d16
                    plsc.addupdate_scatter(acc, [fidx], w16, mask=mlow)
                    plsc.addupdate_scatter(acc, [fidx], w16, mask=mhigh)
                return cc
            lax.fori_loop(0, ECH // (5 * NLANE), deg16, 0)
            return c
        lax.fori_loop(0, ED // ECH, deg_chunk, 0)

        def red(i, c):
            for u in range(4):
                j = 4 * i + u
                s = acc[pl.ds(j * NLANE, NLANE)]
                for p in range(1, 8):
                    s = s + acc[pl.ds(p * NPAD + j * NLANE, NLANE)]
                dis_loc[pl.ds(j * NLANE, NLANE)] = s
            return c
        lax.fori_loop(0, NPAD // (4 * NLANE), red, 0)

        pltpu.sync_copy(dis_loc, sp_part.at[pl.ds(sid * NPAD, NPAD)])
        plsc.subcore_barrier()

        for r in range(NTILE):
            pltpu.sync_copy(sp_part.at[pl.ds(r * NPAD + sid * SLICE, SLICE)],
                            psum.at[pl.ds(r * SLICE, SLICE)])

        magic = jnp.full((NLANE,), 0x5F3759DF, jnp.int32)

        def disloop(i, c):
            s = psum[pl.ds(i * NLANE, NLANE)]
            for r in range(1, NTILE):
                s = s + psum[pl.ds(r * SLICE + i * NLANE, NLANE)]
            d = s + 1.0
            ibits = plsc.bitcast(d, jnp.int32)
            y = plsc.bitcast(magic - lax.shift_right_logical(ibits, 1),
                             jnp.float32)
            for _ in range(3):
                y = y * (1.5 - 0.5 * d * y * y)
            dslice[pl.ds(i * NLANE, NLANE)] = y
            islice[pl.ds(i * NLANE, NLANE)] = y * y
            return c
        lax.fori_loop(0, SLICE // NLANE, disloop, 0)

        pltpu.sync_copy(dslice, sp_dis.at[pl.ds(sid * SLICE, SLICE)])

        @pl.when(cid == 0)
        def _():
            pltpu.sync_copy(islice, inv_hbm.at[pl.ds(sid * SLICE, SLICE)])

        plsc.subcore_barrier()
        pltpu.sync_copy(sp_dis, dis_loc)

        wid = cid * NTILE + sid

        def nchunk(k, c):
            base = wid * EN + k * NCH
            pltpu.sync_copy(src_hbm.at[pl.ds(base, NCH)], sb)
            pltpu.sync_copy(dst_hbm.at[pl.ds(base, NCH)], db)
            pltpu.sync_copy(ew_hbm.at[pl.ds(base, NCH)], wb)

            def n16(i, cc):
                for u in range(5):
                    sl = pl.ds((5 * i + u) * NLANE, NLANE)
                    a = plsc.load_gather(dis_loc, [sb[sl]])
                    b = plsc.load_gather(dis_loc, [db[sl]])
                    nb[sl] = a * wb[sl] * b
                return cc
            lax.fori_loop(0, NCH // (5 * NLANE), n16, 0)
            pltpu.sync_copy(nb, norm_hbm.at[pl.ds(base, NCH)])
            return c
        lax.fori_loop(0, EN // NCH, nchunk, 0)

    return pl.kernel(
        body,
        out_type=[jax.ShapeDtypeStruct((E,), jnp.float32),
                  jax.ShapeDtypeStruct((NPAD,), jnp.float32)],
        mesh=_mesh(),
        compiler_params=pltpu.CompilerParams(**_SC_PARAMS),
        scratch_types=[
            pltpu.VMEM((8 * NPAD,), jnp.float32),
            pltpu.VMEM((NPAD,), jnp.float32),
            pltpu.VMEM((NCH,), jnp.int32),
            pltpu.VMEM((NCH,), jnp.int32),
            pltpu.VMEM((NCH,), jnp.float32),
            pltpu.VMEM((NCH,), jnp.float32),
            pltpu.VMEM((NTILE * SLICE,), jnp.float32),
            pltpu.VMEM((SLICE,), jnp.float32),
            pltpu.VMEM((SLICE,), jnp.float32),
            pltpu.VMEM_SHARED((NTILE * NPAD,), jnp.float32),
            pltpu.VMEM_SHARED((NPAD,), jnp.float32),
        ],
    )


@functools.lru_cache(maxsize=None)
def _make_sc_agg(E, NPAD, F):
    P = F // 8
    GPS = NTILE // P
    NG = NCORE * GPS
    EG = E // NG
    NC = 50
    CH = EG // NC
    SLEN = 400
    NS = CH // SLEN
    NROWS = NPAD // GPS
    RB = 640
    UN = 8
    assert EG % NC == 0 and CH % SLEN == 0 and NROWS % RB == 0
    assert NC % 2 == 0 and (CH // 2) % UN == 0 and CH % (2 * NLANE) == 0

    def body(tab_hbm, src_hbm, dst_hbm, norm_hbm, out_hbm,
             acc, rows0, rows1, rbuf, sb0, sb1, db0, db1, nb0, nb1,
             sp_stage, seml0, seml1, semr0, semr1):
        rows = (rows0, rows1)
        sb = (sb0, sb1)
        db = (db0, db1)
        nb = (nb0, nb1)
        seml = (seml0, seml1)
        semr = (semr0, semr1)
        cid = lax.axis_index("c")
        sid = lax.axis_index("s")
        gl = sid // P
        t = sid - gl * P
        g = cid * GPS + gl
        iota = lax.iota(jnp.int32, NLANE)
        col = jnp.bitwise_and(iota, 7)
        c01 = lax.shift_right_logical(iota, 3)
        c01u = [c01 + 2 * u for u in range(UN)]
        mlow = iota < 8
        mhigh = jnp.logical_not(mlow)
        fzero = jnp.zeros((NLANE,), jnp.float32)
        two = jnp.full((NLANE,), 2, jnp.int32)

        def zloop(i, c):
            for u in range(8):
                plsc.store_scatter(acc, [two * (8 * i + u) + c01, col],
                                   fzero)
            return c
        lax.fori_loop(0, NPAD // 16, zloop, 0)

        toff = jnp.full((NLANE,), 1, jnp.int32) * t

        def lin_issue(k, s):
            base = g * EG + k * CH
            pltpu.async_copy(src_hbm.at[pl.ds(base, CH)], sb[s], seml[s])
            pltpu.async_copy(dst_hbm.at[pl.ds(base, CH)], db[s], seml[s])
            pltpu.async_copy(norm_hbm.at[pl.ds(base, CH)], nb[s], seml[s])

        def lin_wait(s):
            pltpu.make_async_copy(src_hbm.at[pl.ds(0, CH)], sb[s],
                                  seml[s]).wait()
            pltpu.make_async_copy(dst_hbm.at[pl.ds(0, CH)], db[s],
                                  seml[s]).wait()
            pltpu.make_async_copy(norm_hbm.at[pl.ds(0, CH)], nb[s],
                                  seml[s]).wait()

        def adj(s):
            def adj1(i, cc):
                for u in range(2):
                    sl = pl.ds((2 * i + u) * NLANE, NLANE)
                    sb[s][sl] = sb[s][sl] * P + toff
                return cc
            lax.fori_loop(0, CH // (2 * NLANE), adj1, 0)

        def rows_fire(s):
            for j in range(NS):
                pltpu.async_copy(
                    tab_hbm.at[sb[s].at[pl.ds(j * SLEN, SLEN)]],
                    rows[s].at[pl.ds(j * SLEN, SLEN)], semr[s])

        def rows_wait(s):
            for j in range(NS):
                pltpu.make_async_copy(
                    tab_hbm.at[sb[s].at[pl.ds(j * SLEN, SLEN)]],
                    rows[s].at[pl.ds(j * SLEN, SLEN)], semr[s]).wait()

        def compute(s):
            def pairs(q, cc):
                b16 = jnp.full((NLANE,), 2 * UN, jnp.int32) * q
                for u in range(UN):
                    psel = b16 + c01u[u]
                    dstp = plsc.load_gather(db[s], [psel])
                    normp = plsc.load_gather(nb[s], [psel])
                    r16 = plsc.load_gather(rows[s], [psel, col])
                    val = r16 * normp
                    plsc.addupdate_scatter(acc, [dstp, col], val,
                                           mask=mlow)
                    plsc.addupdate_scatter(acc, [dstp, col], val,
                                           mask=mhigh)
                return cc
            lax.fori_loop(0, CH // (2 * UN), pairs, 0)

        lin_issue(0, 0)
        lin_wait(0)
        adj(0)
        rows_fire(0)
        lin_issue(1, 1)

        def piter(ko, c):
            for u2 in range(2):
                k = ko * 2 + u2
                s = u2
                rows_wait(s)

                @pl.when(k + 1 < NC)
                def _():
                    lin_wait(1 - s)
                    adj(1 - s)
                    rows_fire(1 - s)

                compute(s)

                @pl.when(k + 2 < NC)
                def _():
                    lin_issue(k + 2, s)
            return c
        lax.fori_loop(0, NC // 2, piter, 0)

        qbase = gl * NROWS
        for o in range(1, GPS):
            src_tile = ((gl + o) % GPS) * P + t
            give = ((gl + GPS - o) % GPS) * NROWS

            def fold_blk(b, c):
                pltpu.sync_copy(acc.at[pl.ds(give + b * RB, RB)],
                                sp_stage.at[sid])
                plsc.subcore_barrier()
                pltpu.sync_copy(sp_stage.at[src_tile], rbuf)
                rowb = qbase + b * RB

                def fold16(i, cc):
                    for u in range(8):
                        rp = two * (8 * i + u) + c01
                        v = plsc.load_gather(rbuf, [rp, col])
                        tp = rp + rowb
                        plsc.addupdate_scatter(acc, [tp, col], v,
                                               mask=mlow)
                        plsc.addupdate_scatter(acc, [tp, col], v,
                                               mask=mhigh)
                    return cc
                lax.fori_loop(0, RB // 16, fold16, 0)
                plsc.subcore_barrier()
                return c
            lax.fori_loop(0, NROWS // RB, fold_blk, 0)

        pltpu.sync_copy(acc.at[pl.ds(qbase, NROWS)],
                        out_hbm.at[cid, pl.ds(qbase, NROWS),
                                   pl.ds(t * 8, 8)])

    return pl.kernel(
        body,
        out_type=jax.ShapeDtypeStruct((NCORE, NPAD, F), jnp.float32),
        mesh=_mesh(),
        compiler_params=pltpu.CompilerParams(**_SC_PARAMS),
        scratch_types=[
            pltpu.VMEM((NPAD, 8), jnp.float32),
            pltpu.VMEM((CH, 8), jnp.float32),
            pltpu.VMEM((CH, 8), jnp.float32),
            pltpu.VMEM((RB, 8), jnp.float32),
            pltpu.VMEM((CH,), jnp.int32),
            pltpu.VMEM((CH,), jnp.int32),
            pltpu.VMEM((CH,), jnp.int32),
            pltpu.VMEM((CH,), jnp.int32),
            pltpu.VMEM((CH,), jnp.float32),
            pltpu.VMEM((CH,), jnp.float32),
            pltpu.VMEM_SHARED((NTILE, RB, 8), jnp.float32),
            pltpu.SemaphoreType.DMA,
            pltpu.SemaphoreType.DMA,
            pltpu.SemaphoreType.DMA,
            pltpu.SemaphoreType.DMA,
        ],
    )


@functools.lru_cache(maxsize=None)
def _make_tc_pre(N, NPAD, DIN, F):
    def body(x_ref, w_ref, out_ref):
        h = jnp.dot(x_ref[...], w_ref[...],
                    preferred_element_type=jnp.float32)
        out_ref[:N] = h
        out_ref[N:] = jnp.zeros((NPAD - N, F), jnp.float32)

    return pl.pallas_call(
        body,
        out_shape=jax.ShapeDtypeStruct((NPAD, F), jnp.float32))


def _bn(h, g, b):
    m = jnp.mean(h, axis=0, keepdims=True)
    v = jnp.mean((h - m) * (h - m), axis=0, keepdims=True)
    return (h - m) * lax.rsqrt(v + 1e-5) * g + b


@functools.lru_cache(maxsize=None)
def _make_tc_mid(N, NPAD, F1, F2):
    def body(part_ref, ht_ref, inv_ref, b_ref, g_ref, be_ref, w_ref,
             out_ref):
        inv = inv_ref[:N]
        h = (part_ref[0, :N, :] + part_ref[1, :N, :]
             + inv * ht_ref[:N, :] + b_ref[0])
        h = jnp.maximum(_bn(h, g_ref[0], be_ref[0]), 0.0)
        h2 = jnp.dot(h, w_ref[...], preferred_element_type=jnp.float32)
        out_ref[:N] = h2
        out_ref[N:] = jnp.zeros((NPAD - N, F2), jnp.float32)

    return pl.pallas_call(
        body,
        out_shape=jax.ShapeDtypeStruct((NPAD, F2), jnp.float32))


@functools.lru_cache(maxsize=None)
def _make_tc_post(N, NPAD, F2, FD1, DOUT):
    def body(part_ref, ht_ref, inv_ref, b2_ref, g2_ref, be2_ref,
             wd1_ref, bd1_ref, gd1_ref, bed1_ref, wd2_ref, bd2_ref,
             z_ref, recon_ref):
        inv = inv_ref[:N]
        h = (part_ref[0, :N, :] + part_ref[1, :N, :]
             + inv * ht_ref[:N, :] + b2_ref[0])
        z = jnp.maximum(_bn(h, g2_ref[0], be2_ref[0]), 0.0)
        z_ref[...] = z
        d = jnp.dot(z, wd1_ref[...],
                    preferred_element_type=jnp.float32) + bd1_ref[0]
        d = jnp.maximum(_bn(d, gd1_ref[0], bed1_ref[0]), 0.0)
        recon_ref[...] = jnp.dot(d, wd2_ref[...],
                                 preferred_element_type=jnp.float32) + bd2_ref[0]

    return pl.pallas_call(
        body,
        out_shape=[jax.ShapeDtypeStruct((N, F2), jnp.float32),
                   jax.ShapeDtypeStruct((N, DOUT), jnp.float32)])


def kernel(x, edge_index, edge_attr, W1, b1, g1, be1, W2, b2, g2, be2,
           Wd1, bd1, gd1, bed1, Wd2, bd2):
    N, DIN = x.shape
    E = edge_attr.shape[0]
    F1, F2 = W1.shape[1], W2.shape[1]
    FD1, DOUT = Wd1.shape[1], Wd2.shape[1]
    NPAD = ((N + 255) // 256) * 256

    src = edge_index[0]
    dst = edge_index[1]

    norm, invp = _make_sc_norm(E, NPAD)(src, dst, edge_attr)
    inv2 = invp.reshape(NPAD, 1)

    h1 = _make_tc_pre(N, NPAD, DIN, F1)(x, W1)
    part1 = _make_sc_agg(E, NPAD, F1)(
        h1.reshape(NPAD * (F1 // 8), 8), src, dst, norm)

    h2 = _make_tc_mid(N, NPAD, F1, F2)(
        part1, h1, inv2, b1.reshape(1, F1), g1.reshape(1, F1),
        be1.reshape(1, F1), W2)
    part2 = _make_sc_agg(E, NPAD, F2)(
        h2.reshape(NPAD * (F2 // 8), 8), src, dst, norm)

    z, recon = _make_tc_post(N, NPAD, F2, FD1, DOUT)(
        part2, h2, inv2, b2.reshape(1, F2), g2.reshape(1, F2),
        be2.reshape(1, F2), Wd1, bd1.reshape(1, FD1), gd1.reshape(1, FD1),
        bed1.reshape(1, FD1), Wd2, bd2.reshape(1, DOUT))
    return (recon, z)


# speedup vs baseline: 12.0731x; 1.1570x over previous
import functools

import jax
import jax.numpy as jnp
from jax import lax
from jax.experimental import pallas as pl
from jax.experimental.pallas import tpu as pltpu
from jax.experimental.pallas import tpu_sc as plsc

NLANE = 16
NTILE = 16
NCORE = 2
NW = NCORE * NTILE


def _mesh():
    return plsc.VectorSubcoreMesh(core_axis_name="c", subcore_axis_name="s")


_SC_PARAMS = dict(needs_layout_passes=False, use_tc_tiling_on_sc=False)


@functools.lru_cache(maxsize=None)
def _make_sc_norm(E, NPAD):
    SLICE = NPAD // NTILE
    ED = E // NTILE
    ECH = 2000
    EN = E // NW
    NCH = 2000
    assert ED % ECH == 0 and EN % NCH == 0 and SLICE % NLANE == 0

    def body(src_hbm, dst_hbm, ew_hbm, norm_hbm, inv_hbm,
             acc, dis_loc, sb, db, wb, nb, psum, dslice, islice,
             sp_part, sp_dis):
        cid = lax.axis_index("c")
        sid = lax.axis_index("s")
        iota = lax.iota(jnp.int32, NLANE)
        plane = jnp.bitwise_and(iota, 7)
        mlow = iota < 8
        mhigh = jnp.logical_not(mlow)
        fzero = jnp.zeros((NLANE,), jnp.float32)

        def zloop(i, c):
            for u in range(8):
                acc[pl.ds((8 * i + u) * NLANE, NLANE)] = fzero
            return c
        lax.fori_loop(0, (8 * NPAD) // (8 * NLANE), zloop, 0)

        def deg_chunk(k, c):
            base = sid * ED + k * ECH
            pltpu.sync_copy(dst_hbm.at[pl.ds(base, ECH)], db)
            pltpu.sync_copy(ew_hbm.at[pl.ds(base, ECH)], wb)

            def deg16(i, cc):
                for u in range(5):
                    sl = pl.ds((5 * i + u) * NLANE, NLANE)
                    d16 = db[sl]
                    w16 = wb[sl]
                    fidx = plane * NPAD + d16
                    plsc.addupdate_scatter(acc, [fidx], w16, mask=mlow)
                    plsc.addupdate_scatter(acc, [fidx], w16, mask=mhigh)
                return cc
            lax.fori_loop(0, ECH // (5 * NLANE), deg16, 0)
            return c
        lax.fori_loop(0, ED // ECH, deg_chunk, 0)

        def red(i, c):
            for u in range(4):
                j = 4 * i + u
                s = acc[pl.ds(j * NLANE, NLANE)]
                for p in range(1, 8):
                    s = s + acc[pl.ds(p * NPAD + j * NLANE, NLANE)]
                dis_loc[pl.ds(j * NLANE, NLANE)] = s
            return c
        lax.fori_loop(0, NPAD // (4 * NLANE), red, 0)

        pltpu.sync_copy(dis_loc, sp_part.at[pl.ds(sid * NPAD, NPAD)])
        plsc.subcore_barrier()

        for r in range(NTILE):
            pltpu.sync_copy(sp_part.at[pl.ds(r * NPAD + sid * SLICE, SLICE)],
                            psum.at[pl.ds(r * SLICE, SLICE)])

        magic = jnp.full((NLANE,), 0x5F3759DF, jnp.int32)

        def disloop(i, c):
            s = psum[pl.ds(i * NLANE, NLANE)]
            for r in range(1, NTILE):
                s = s + psum[pl.ds(r * SLICE + i * NLANE, NLANE)]
            d = s + 1.0
            ibits = plsc.bitcast(d, jnp.int32)
            y = plsc.bitcast(magic - lax.shift_right_logical(ibits, 1),
                             jnp.float32)
            for _ in range(3):
                y = y * (1.5 - 0.5 * d * y * y)
            dslice[pl.ds(i * NLANE, NLANE)] = y
            islice[pl.ds(i * NLANE, NLANE)] = y * y
            return c
        lax.fori_loop(0, SLICE // NLANE, disloop, 0)

        pltpu.sync_copy(dslice, sp_dis.at[pl.ds(sid * SLICE, SLICE)])

        @pl.when(cid == 0)
        def _():
            pltpu.sync_copy(islice, inv_hbm.at[pl.ds(sid * SLICE, SLICE)])

        plsc.subcore_barrier()
        pltpu.sync_copy(sp_dis, dis_loc)

        wid = cid * NTILE + sid

        def nchunk(k, c):
            base = wid * EN + k * NCH
            pltpu.sync_copy(src_hbm.at[pl.ds(base, NCH)], sb)
            pltpu.sync_copy(dst_hbm.at[pl.ds(base, NCH)], db)
            pltpu.sync_copy(ew_hbm.at[pl.ds(base, NCH)], wb)

            def n16(i, cc):
                for u in range(5):
                    sl = pl.ds((5 * i + u) * NLANE, NLANE)
                    a = plsc.load_gather(dis_loc, [sb[sl]])
                    b = plsc.load_gather(dis_loc, [db[sl]])
                    nb[sl] = a * wb[sl] * b
                return cc
            lax.fori_loop(0, NCH // (5 * NLANE), n16, 0)
            pltpu.sync_copy(nb, norm_hbm.at[pl.ds(base, NCH)])
            return c
        lax.fori_loop(0, EN // NCH, nchunk, 0)

    return pl.kernel(
        body,
        out_type=[jax.ShapeDtypeStruct((E,), jnp.float32),
                  jax.ShapeDtypeStruct((NPAD,), jnp.float32)],
        mesh=_mesh(),
        compiler_params=pltpu.CompilerParams(**_SC_PARAMS),
        scratch_types=[
            pltpu.VMEM((8 * NPAD,), jnp.float32),
            pltpu.VMEM((NPAD,), jnp.float32),
            pltpu.VMEM((NCH,), jnp.int32),
            pltpu.VMEM((NCH,), jnp.int32),
            pltpu.VMEM((NCH,), jnp.float32),
            pltpu.VMEM((NCH,), jnp.float32),
            pltpu.VMEM((NTILE * SLICE,), jnp.float32),
            pltpu.VMEM((SLICE,), jnp.float32),
            pltpu.VMEM((SLICE,), jnp.float32),
            pltpu.VMEM_SHARED((NTILE * NPAD,), jnp.float32),
            pltpu.VMEM_SHARED((NPAD,), jnp.float32),
        ],
    )


@functools.lru_cache(maxsize=None)
def _make_sc_agg(E, NPAD, F):
    P = F // 8
    GPS = NTILE // P
    NG = NCORE * GPS
    EG = E // NG
    NC = 50
    CH = EG // NC
    SLEN = 400
    NS = CH // SLEN
    NROWS = NPAD // GPS
    RB = 640
    UN = 8
    assert EG % NC == 0 and CH % SLEN == 0 and NROWS % RB == 0
    assert NC % 2 == 0 and (CH // 2) % UN == 0 and CH % (2 * NLANE) == 0

    def body(tab_hbm, src_hbm, dst_hbm, norm_hbm, out_hbm,
             acc, rows0, rows1, rbuf, sb0, sb1, db0, db1, nb0, nb1,
             sp_stage, seml0, seml1, semr0, semr1):
        rows = (rows0, rows1)
        sb = (sb0, sb1)
        db = (db0, db1)
        nb = (nb0, nb1)
        seml = (seml0, seml1)
        semr = (semr0, semr1)
        cid = lax.axis_index("c")
        sid = lax.axis_index("s")
        gl = sid // P
        t = sid - gl * P
        g = cid * GPS + gl
        iota = lax.iota(jnp.int32, NLANE)
        col = jnp.bitwise_and(iota, 7)
        c01 = lax.shift_right_logical(iota, 3)
        c01u = [c01 + 2 * u for u in range(UN)]
        mlow = iota < 8
        mhigh = jnp.logical_not(mlow)
        fzero = jnp.zeros((NLANE,), jnp.float32)
        two = jnp.full((NLANE,), 2, jnp.int32)

        def zloop(i, c):
            for u in range(8):
                plsc.store_scatter(acc, [two * (8 * i + u) + c01, col],
                                   fzero)
            return c
        lax.fori_loop(0, NPAD // 16, zloop, 0)

        toff = jnp.full((NLANE,), 1, jnp.int32) * t

        def lin_issue(k, s):
            base = g * EG + k * CH
            pltpu.async_copy(src_hbm.at[pl.ds(base, CH)], sb[s], seml[s])
            pltpu.async_copy(dst_hbm.at[pl.ds(base, CH)], db[s], seml[s])
            pltpu.async_copy(norm_hbm.at[pl.ds(base, CH)], nb[s], seml[s])

        def lin_wait(s):
            pltpu.make_async_copy(src_hbm.at[pl.ds(0, CH)], sb[s],
                                  seml[s]).wait()
            pltpu.make_async_copy(dst_hbm.at[pl.ds(0, CH)], db[s],
                                  seml[s]).wait()
            pltpu.make_async_copy(norm_hbm.at[pl.ds(0, CH)], nb[s],
                                  seml[s]).wait()

        def adj(s):
            def adj1(i, cc):
                for u in range(2):
                    sl = pl.ds((2 * i + u) * NLANE, NLANE)
                    sb[s][sl] = sb[s][sl] * P + toff
                return cc
            lax.fori_loop(0, CH // (2 * NLANE), adj1, 0)

        def rows_fire(s):
            for j in range(NS):
                pltpu.async_copy(
                    tab_hbm.at[sb[s].at[pl.ds(j * SLEN, SLEN)]],
                    rows[s].at[pl.ds(j * SLEN, SLEN)], semr[s])

        def rows_wait(s):
            for j in range(NS):
                pltpu.make_async_copy(
                    tab_hbm.at[sb[s].at[pl.ds(j * SLEN, SLEN)]],
                    rows[s].at[pl.ds(j * SLEN, SLEN)], semr[s]).wait()

        def compute(s):
            def pairs(q, cc):
                base = q * (2 * UN)
                d16 = db[s][pl.ds(base, 2 * UN)]
                n16 = nb[s][pl.ds(base, 2 * UN)]
                b16 = jnp.full((NLANE,), 2 * UN, jnp.int32) * q
                for u in range(UN):
                    dstp = d16.at[c01u[u]].get(mode="promise_in_bounds")
                    normp = n16.at[c01u[u]].get(mode="promise_in_bounds")
                    psel = b16 + c01u[u]
                    r16 = plsc.load_gather(rows[s], [psel, col])
                    val = r16 * normp
                    plsc.addupdate_scatter(acc, [dstp, col], val,
                                           mask=mlow)
                    plsc.addupdate_scatter(acc, [dstp, col], val,
                                           mask=mhigh)
                return cc
            lax.fori_loop(0, CH // (2 * UN), pairs, 0)

        lin_issue(0, 0)
        lin_wait(0)
        adj(0)
        rows_fire(0)
        lin_issue(1, 1)

        def piter(ko, c):
            for u2 in range(2):
                k = ko * 2 + u2
                s = u2
                rows_wait(s)

                @pl.when(k + 1 < NC)
                def _():
                    lin_wait(1 - s)
                    adj(1 - s)
                    rows_fire(1 - s)

                compute(s)

                @pl.when(k + 2 < NC)
                def _():
                    lin_issue(k + 2, s)
            return c
        lax.fori_loop(0, NC // 2, piter, 0)

        qbase = gl * NROWS
        for o in range(1, GPS):
            src_tile = ((gl + o) % GPS) * P + t
            give = ((gl + GPS - o) % GPS) * NROWS

            def fold_blk(b, c):
                pltpu.sync_copy(acc.at[pl.ds(give + b * RB, RB)],
                                sp_stage.at[sid])
                plsc.subcore_barrier()
                pltpu.sync_copy(sp_stage.at[src_tile], rbuf)
                rowb = qbase + b * RB

                def fold16(i, cc):
                    for u in range(8):
                        rp = two * (8 * i + u) + c01
                        v = plsc.load_gather(rbuf, [rp, col])
                        tp = rp + rowb
                        plsc.addupdate_scatter(acc, [tp, col], v,
                                               mask=mlow)
                        plsc.addupdate_scatter(acc, [tp, col], v,
                                               mask=mhigh)
                    return cc
                lax.fori_loop(0, RB // 16, fold16, 0)
                plsc.subcore_barrier()
                return c
            lax.fori_loop(0, NROWS // RB, fold_blk, 0)

        pltpu.sync_copy(acc.at[pl.ds(qbase, NROWS)],
                        out_hbm.at[cid, pl.ds(qbase, NROWS),
                                   pl.ds(t * 8, 8)])

    return pl.kernel(
        body,
        out_type=jax.ShapeDtypeStruct((NCORE, NPAD, F), jnp.float32),
        mesh=_mesh(),
        compiler_params=pltpu.CompilerParams(**_SC_PARAMS),
        scratch_types=[
            pltpu.VMEM((NPAD, 8), jnp.float32),
            pltpu.VMEM((CH, 8), jnp.float32),
            pltpu.VMEM((CH, 8), jnp.float32),
            pltpu.VMEM((RB, 8), jnp.float32),
            pltpu.VMEM((CH,), jnp.int32),
            pltpu.VMEM((CH,), jnp.int32),
            pltpu.VMEM((CH,), jnp.int32),
            pltpu.VMEM((CH,), jnp.int32),
            pltpu.VMEM((CH,), jnp.float32),
            pltpu.VMEM((CH,), jnp.float32),
            pltpu.VMEM_SHARED((NTILE, RB, 8), jnp.float32),
            pltpu.SemaphoreType.DMA,
            pltpu.SemaphoreType.DMA,
            pltpu.SemaphoreType.DMA,
            pltpu.SemaphoreType.DMA,
        ],
    )


@functools.lru_cache(maxsize=None)
def _make_tc_pre(N, NPAD, DIN, F):
    def body(x_ref, w_ref, out_ref):
        h = jnp.dot(x_ref[...], w_ref[...],
                    preferred_element_type=jnp.float32)
        out_ref[:N] = h
        out_ref[N:] = jnp.zeros((NPAD - N, F), jnp.float32)

    return pl.pallas_call(
        body,
        out_shape=jax.ShapeDtypeStruct((NPAD, F), jnp.float32))


def _bn(h, g, b):
    m = jnp.mean(h, axis=0, keepdims=True)
    v = jnp.mean((h - m) * (h - m), axis=0, keepdims=True)
    return (h - m) * lax.rsqrt(v + 1e-5) * g + b


@functools.lru_cache(maxsize=None)
def _make_tc_mid(N, NPAD, F1, F2):
    def body(part_ref, ht_ref, inv_ref, b_ref, g_ref, be_ref, w_ref,
             out_ref):
        inv = inv_ref[:N]
        h = (part_ref[0, :N, :] + part_ref[1, :N, :]
             + inv * ht_ref[:N, :] + b_ref[0])
        h = jnp.maximum(_bn(h, g_ref[0], be_ref[0]), 0.0)
        h2 = jnp.dot(h, w_ref[...], preferred_element_type=jnp.float32)
        out_ref[:N] = h2
        out_ref[N:] = jnp.zeros((NPAD - N, F2), jnp.float32)

    return pl.pallas_call(
        body,
        out_shape=jax.ShapeDtypeStruct((NPAD, F2), jnp.float32))


@functools.lru_cache(maxsize=None)
def _make_tc_post(N, NPAD, F2, FD1, DOUT):
    def body(part_ref, ht_ref, inv_ref, b2_ref, g2_ref, be2_ref,
             wd1_ref, bd1_ref, gd1_ref, bed1_ref, wd2_ref, bd2_ref,
             z_ref, recon_ref):
        inv = inv_ref[:N]
        h = (part_ref[0, :N, :] + part_ref[1, :N, :]
             + inv * ht_ref[:N, :] + b2_ref[0])
        z = jnp.maximum(_bn(h, g2_ref[0], be2_ref[0]), 0.0)
        z_ref[...] = z
        d = jnp.dot(z, wd1_ref[...],
                    preferred_element_type=jnp.float32) + bd1_ref[0]
        d = jnp.maximum(_bn(d, gd1_ref[0], bed1_ref[0]), 0.0)
        recon_ref[...] = jnp.dot(d, wd2_ref[...],
                                 preferred_element_type=jnp.float32) + bd2_ref[0]

    return pl.pallas_call(
        body,
        out_shape=[jax.ShapeDtypeStruct((N, F2), jnp.float32),
                   jax.ShapeDtypeStruct((N, DOUT), jnp.float32)])


def kernel(x, edge_index, edge_attr, W1, b1, g1, be1, W2, b2, g2, be2,
           Wd1, bd1, gd1, bed1, Wd2, bd2):
    N, DIN = x.shape
    E = edge_attr.shape[0]
    F1, F2 = W1.shape[1], W2.shape[1]
    FD1, DOUT = Wd1.shape[1], Wd2.shape[1]
    NPAD = ((N + 255) // 256) * 256

    src = edge_index[0]
    dst = edge_index[1]

    norm, invp = _make_sc_norm(E, NPAD)(src, dst, edge_attr)
    inv2 = invp.reshape(NPAD, 1)

    h1 = _make_tc_pre(N, NPAD, DIN, F1)(x, W1)
    part1 = _make_sc_agg(E, NPAD, F1)(
        h1.reshape(NPAD * (F1 // 8), 8), src, dst, norm)

    h2 = _make_tc_mid(N, NPAD, F1, F2)(
        part1, h1, inv2, b1.reshape(1, F1), g1.reshape(1, F1),
        be1.reshape(1, F1), W2)
    part2 = _make_sc_agg(E, NPAD, F2)(
        h2.reshape(NPAD * (F2 // 8), 8), src, dst, norm)

    z, recon = _make_tc_post(N, NPAD, F2, FD1, DOUT)(
        part2, h2, inv2, b2.reshape(1, F2), g2.reshape(1, F2),
        be2.reshape(1, F2), Wd1, bd1.reshape(1, FD1), gd1.reshape(1, FD1),
        bed1.reshape(1, FD1), Wd2, bd2.reshape(1, DOUT))
    return (recon, z)


# speedup vs baseline: 12.9139x; 1.0696x over previous
import functools

import jax
import jax.numpy as jnp
from jax import lax
from jax.experimental import pallas as pl
from jax.experimental.pallas import tpu as pltpu
from jax.experimental.pallas import tpu_sc as plsc

NLANE = 16
NTILE = 16
NCORE = 2
NW = NCORE * NTILE


def _mesh():
    return plsc.VectorSubcoreMesh(core_axis_name="c", subcore_axis_name="s")


_SC_PARAMS = dict(needs_layout_passes=False, use_tc_tiling_on_sc=False)


@functools.lru_cache(maxsize=None)
def _make_sc_norm(E, NPAD):
    SLICE = NPAD // NTILE
    ED = E // NTILE
    ECH = 2000
    EN = E // NW
    NCH = 2000
    assert ED % ECH == 0 and EN % NCH == 0 and SLICE % NLANE == 0

    def body(src_hbm, dst_hbm, ew_hbm, norm_hbm, inv_hbm,
             acc, dis_loc, sb, db, wb, nb, psum, dslice, islice,
             sp_part, sp_dis):
        cid = lax.axis_index("c")
        sid = lax.axis_index("s")
        fzero = jnp.zeros((NLANE,), jnp.float32)

        def zloop(i, c):
            for u in range(8):
                acc[pl.ds((8 * i + u) * NLANE, NLANE)] = fzero
            return c
        lax.fori_loop(0, NPAD // (8 * NLANE), zloop, 0)

        def deg_chunk(k, c):
            base = sid * ED + k * ECH
            pltpu.sync_copy(dst_hbm.at[pl.ds(base, ECH)], db)
            pltpu.sync_copy(ew_hbm.at[pl.ds(base, ECH)], wb)

            def deg16(i, cc):
                for u in range(5):
                    sl = pl.ds((5 * i + u) * NLANE, NLANE)
                    plsc.addupdate_scatter(acc, [db[sl]], wb[sl])
                return cc
            lax.fori_loop(0, ECH // (5 * NLANE), deg16, 0)
            return c
        lax.fori_loop(0, ED // ECH, deg_chunk, 0)

        pltpu.sync_copy(acc, sp_part.at[pl.ds(sid * NPAD, NPAD)])
        plsc.subcore_barrier()

        for r in range(NTILE):
            pltpu.sync_copy(sp_part.at[pl.ds(r * NPAD + sid * SLICE, SLICE)],
                            psum.at[pl.ds(r * SLICE, SLICE)])

        magic = jnp.full((NLANE,), 0x5F3759DF, jnp.int32)

        def disloop(i, c):
            s = psum[pl.ds(i * NLANE, NLANE)]
            for r in range(1, NTILE):
                s = s + psum[pl.ds(r * SLICE + i * NLANE, NLANE)]
            d = s + 1.0
            ibits = plsc.bitcast(d, jnp.int32)
            y = plsc.bitcast(magic - lax.shift_right_logical(ibits, 1),
                             jnp.float32)
            for _ in range(3):
                y = y * (1.5 - 0.5 * d * y * y)
            dslice[pl.ds(i * NLANE, NLANE)] = y
            islice[pl.ds(i * NLANE, NLANE)] = y * y
            return c
        lax.fori_loop(0, SLICE // NLANE, disloop, 0)

        pltpu.sync_copy(dslice, sp_dis.at[pl.ds(sid * SLICE, SLICE)])

        @pl.when(cid == 0)
        def _():
            pltpu.sync_copy(islice, inv_hbm.at[pl.ds(sid * SLICE, SLICE)])

        plsc.subcore_barrier()
        pltpu.sync_copy(sp_dis, dis_loc)

        wid = cid * NTILE + sid

        def nchunk(k, c):
            base = wid * EN + k * NCH
            pltpu.sync_copy(src_hbm.at[pl.ds(base, NCH)], sb)
            pltpu.sync_copy(dst_hbm.at[pl.ds(base, NCH)], db)
            pltpu.sync_copy(ew_hbm.at[pl.ds(base, NCH)], wb)

            def n16(i, cc):
                for u in range(5):
                    sl = pl.ds((5 * i + u) * NLANE, NLANE)
                    a = plsc.load_gather(dis_loc, [sb[sl]])
                    b = plsc.load_gather(dis_loc, [db[sl]])
                    nb[sl] = a * wb[sl] * b
                return cc
            lax.fori_loop(0, NCH // (5 * NLANE), n16, 0)
            pltpu.sync_copy(nb, norm_hbm.at[pl.ds(base, NCH)])
            return c
        lax.fori_loop(0, EN // NCH, nchunk, 0)

    return pl.kernel(
        body,
        out_type=[jax.ShapeDtypeStruct((E,), jnp.float32),
                  jax.ShapeDtypeStruct((NPAD,), jnp.float32)],
        mesh=_mesh(),
        compiler_params=pltpu.CompilerParams(**_SC_PARAMS),
        scratch_types=[
            pltpu.VMEM((NPAD,), jnp.float32),
            pltpu.VMEM((NPAD,), jnp.float32),
            pltpu.VMEM((NCH,), jnp.int32),
            pltpu.VMEM((NCH,), jnp.int32),
            pltpu.VMEM((NCH,), jnp.float32),
            pltpu.VMEM((NCH,), jnp.float32),
            pltpu.VMEM((NTILE * SLICE,), jnp.float32),
            pltpu.VMEM((SLICE,), jnp.float32),
            pltpu.VMEM((SLICE,), jnp.float32),
            pltpu.VMEM_SHARED((NTILE * NPAD,), jnp.float32),
            pltpu.VMEM_SHARED((NPAD,), jnp.float32),
        ],
    )


@functools.lru_cache(maxsize=None)
def _make_sc_agg(E, NPAD, F):
    P = F // 8
    GPS = NTILE // P
    NG = NCORE * GPS
    EG = E // NG
    NC = 50
    CH = EG // NC
    SLEN = 400
    NS = CH // SLEN
    NROWS = NPAD // GPS
    RB = 640
    UN = 8
    assert EG % NC == 0 and CH % SLEN == 0 and NROWS % RB == 0
    assert NC % 2 == 0 and (CH // 2) % UN == 0 and CH % (2 * NLANE) == 0

    def body(tab_hbm, src_hbm, dst_hbm, norm_hbm, out_hbm,
             acc, rows0, rows1, rbuf, sb0, sb1, db0, db1, nb0, nb1,
             sp_stage, seml0, seml1, semr0, semr1):
        rows = (rows0, rows1)
        sb = (sb0, sb1)
        db = (db0, db1)
        nb = (nb0, nb1)
        seml = (seml0, seml1)
        semr = (semr0, semr1)
        cid = lax.axis_index("c")
        sid = lax.axis_index("s")
        gl = sid // P
        t = sid - gl * P
        g = cid * GPS + gl
        iota = lax.iota(jnp.int32, NLANE)
        col = jnp.bitwise_and(iota, 7)
        c01 = lax.shift_right_logical(iota, 3)
        c01u = [c01 + 2 * u for u in range(UN)]
        mlow = iota < 8
        mhigh = jnp.logical_not(mlow)
        fzero = jnp.zeros((NLANE,), jnp.float32)
        two = jnp.full((NLANE,), 2, jnp.int32)

        def zloop(i, c):
            for u in range(8):
                plsc.store_scatter(acc, [two * (8 * i + u) + c01, col],
                                   fzero)
            return c
        lax.fori_loop(0, NPAD // 16, zloop, 0)

        toff = jnp.full((NLANE,), 1, jnp.int32) * t

        def lin_issue(k, s):
            base = g * EG + k * CH
            pltpu.async_copy(src_hbm.at[pl.ds(base, CH)], sb[s], seml[s])
            pltpu.async_copy(dst_hbm.at[pl.ds(base, CH)], db[s], seml[s])
            pltpu.async_copy(norm_hbm.at[pl.ds(base, CH)], nb[s], seml[s])

        def lin_wait(s):
            pltpu.make_async_copy(src_hbm.at[pl.ds(0, CH)], sb[s],
                                  seml[s]).wait()
            pltpu.make_async_copy(dst_hbm.at[pl.ds(0, CH)], db[s],
                                  seml[s]).wait()
            pltpu.make_async_copy(norm_hbm.at[pl.ds(0, CH)], nb[s],
                                  seml[s]).wait()

        def adj(s):
            def adj1(i, cc):
                for u in range(2):
                    sl = pl.ds((2 * i + u) * NLANE, NLANE)
                    sb[s][sl] = sb[s][sl] * P + toff
                return cc
            lax.fori_loop(0, CH // (2 * NLANE), adj1, 0)

        def rows_fire(s):
            for j in range(NS):
                pltpu.async_copy(
                    tab_hbm.at[sb[s].at[pl.ds(j * SLEN, SLEN)]],
                    rows[s].at[pl.ds(j * SLEN, SLEN)], semr[s])

        def rows_wait(s):
            for j in range(NS):
                pltpu.make_async_copy(
                    tab_hbm.at[sb[s].at[pl.ds(j * SLEN, SLEN)]],
                    rows[s].at[pl.ds(j * SLEN, SLEN)], semr[s]).wait()

        def compute(s):
            def pairs(q, cc):
                base = q * (2 * UN)
                d16 = db[s][pl.ds(base, 2 * UN)]
                n16 = nb[s][pl.ds(base, 2 * UN)]
                b16 = jnp.full((NLANE,), 2 * UN, jnp.int32) * q
                for u in range(UN):
                    dstp = d16.at[c01u[u]].get(mode="promise_in_bounds")
                    normp = n16.at[c01u[u]].get(mode="promise_in_bounds")
                    psel = b16 + c01u[u]
                    r16 = plsc.load_gather(rows[s], [psel, col])
                    val = r16 * normp
                    plsc.addupdate_scatter(acc, [dstp, col], val)
                return cc
            lax.fori_loop(0, CH // (2 * UN), pairs, 0)

        lin_issue(0, 0)
        lin_wait(0)
        adj(0)
        rows_fire(0)
        lin_issue(1, 1)

        def piter(ko, c):
            for u2 in range(2):
                k = ko * 2 + u2
                s = u2
                rows_wait(s)

                @pl.when(k + 1 < NC)
                def _():
                    lin_wait(1 - s)
                    adj(1 - s)
                    rows_fire(1 - s)

                compute(s)

                @pl.when(k + 2 < NC)
                def _():
                    lin_issue(k + 2, s)
            return c
        lax.fori_loop(0, NC // 2, piter, 0)

        qbase = gl * NROWS
        for o in range(1, GPS):
            src_tile = ((gl + o) % GPS) * P + t
            give = ((gl + GPS - o) % GPS) * NROWS

            def fold_blk(b, c):
                pltpu.sync_copy(acc.at[pl.ds(give + b * RB, RB)],
                                sp_stage.at[sid])
                plsc.subcore_barrier()
                pltpu.sync_copy(sp_stage.at[src_tile], rbuf)
                rowb = qbase + b * RB

                def fold16(i, cc):
                    for u in range(8):
                        rp = two * (8 * i + u) + c01
                        v = plsc.load_gather(rbuf, [rp, col])
                        plsc.addupdate_scatter(acc, [rp + rowb, col], v)
                    return cc
                lax.fori_loop(0, RB // 16, fold16, 0)
                plsc.subcore_barrier()
                return c
            lax.fori_loop(0, NROWS // RB, fold_blk, 0)

        pltpu.sync_copy(acc.at[pl.ds(qbase, NROWS)],
                        out_hbm.at[cid, pl.ds(qbase, NROWS),
                                   pl.ds(t * 8, 8)])

    return pl.kernel(
        body,
        out_type=jax.ShapeDtypeStruct((NCORE, NPAD, F), jnp.float32),
        mesh=_mesh(),
        compiler_params=pltpu.CompilerParams(**_SC_PARAMS),
        scratch_types=[
            pltpu.VMEM((NPAD, 8), jnp.float32),
            pltpu.VMEM((CH, 8), jnp.float32),
            pltpu.VMEM((CH, 8), jnp.float32),
            pltpu.VMEM((RB, 8), jnp.float32),
            pltpu.VMEM((CH,), jnp.int32),
            pltpu.VMEM((CH,), jnp.int32),
            pltpu.VMEM((CH,), jnp.int32),
            pltpu.VMEM((CH,), jnp.int32),
            pltpu.VMEM((CH,), jnp.float32),
            pltpu.VMEM((CH,), jnp.float32),
            pltpu.VMEM_SHARED((NTILE, RB, 8), jnp.float32),
            pltpu.SemaphoreType.DMA,
            pltpu.SemaphoreType.DMA,
            pltpu.SemaphoreType.DMA,
            pltpu.SemaphoreType.DMA,
        ],
    )


@functools.lru_cache(maxsize=None)
def _make_tc_pre(N, NPAD, DIN, F):
    def body(x_ref, w_ref, out_ref):
        h = jnp.dot(x_ref[...], w_ref[...],
                    preferred_element_type=jnp.float32)
        out_ref[:N] = h
        out_ref[N:] = jnp.zeros((NPAD - N, F), jnp.float32)

    return pl.pallas_call(
        body,
        out_shape=jax.ShapeDtypeStruct((NPAD, F), jnp.float32))


def _bn(h, g, b):
    m = jnp.mean(h, axis=0, keepdims=True)
    v = jnp.mean((h - m) * (h - m), axis=0, keepdims=True)
    return (h - m) * lax.rsqrt(v + 1e-5) * g + b


@functools.lru_cache(maxsize=None)
def _make_tc_mid(N, NPAD, F1, F2):
    def body(part_ref, ht_ref, inv_ref, b_ref, g_ref, be_ref, w_ref,
             out_ref):
        inv = inv_ref[:N]
        h = (part_ref[0, :N, :] + part_ref[1, :N, :]
             + inv * ht_ref[:N, :] + b_ref[0])
        h = jnp.maximum(_bn(h, g_ref[0], be_ref[0]), 0.0)
        h2 = jnp.dot(h, w_ref[...], preferred_element_type=jnp.float32)
        out_ref[:N] = h2
        out_ref[N:] = jnp.zeros((NPAD - N, F2), jnp.float32)

    return pl.pallas_call(
        body,
        out_shape=jax.ShapeDtypeStruct((NPAD, F2), jnp.float32))


@functools.lru_cache(maxsize=None)
def _make_tc_post(N, NPAD, F2, FD1, DOUT):
    def body(part_ref, ht_ref, inv_ref, b2_ref, g2_ref, be2_ref,
             wd1_ref, bd1_ref, gd1_ref, bed1_ref, wd2_ref, bd2_ref,
             z_ref, recon_ref):
        inv = inv_ref[:N]
        h = (part_ref[0, :N, :] + part_ref[1, :N, :]
             + inv * ht_ref[:N, :] + b2_ref[0])
        z = jnp.maximum(_bn(h, g2_ref[0], be2_ref[0]), 0.0)
        z_ref[...] = z
        d = jnp.dot(z, wd1_ref[...],
                    preferred_element_type=jnp.float32) + bd1_ref[0]
        d = jnp.maximum(_bn(d, gd1_ref[0], bed1_ref[0]), 0.0)
        recon_ref[...] = jnp.dot(d, wd2_ref[...],
                                 preferred_element_type=jnp.float32) + bd2_ref[0]

    return pl.pallas_call(
        body,
        out_shape=[jax.ShapeDtypeStruct((N, F2), jnp.float32),
                   jax.ShapeDtypeStruct((N, DOUT), jnp.float32)])


def kernel(x, edge_index, edge_attr, W1, b1, g1, be1, W2, b2, g2, be2,
           Wd1, bd1, gd1, bed1, Wd2, bd2):
    N, DIN = x.shape
    E = edge_attr.shape[0]
    F1, F2 = W1.shape[1], W2.shape[1]
    FD1, DOUT = Wd1.shape[1], Wd2.shape[1]
    NPAD = ((N + 255) // 256) * 256

    src = edge_index[0]
    dst = edge_index[1]

    norm, invp = _make_sc_norm(E, NPAD)(src, dst, edge_attr)
    inv2 = invp.reshape(NPAD, 1)

    h1 = _make_tc_pre(N, NPAD, DIN, F1)(x, W1)
    part1 = _make_sc_agg(E, NPAD, F1)(
        h1.reshape(NPAD * (F1 // 8), 8), src, dst, norm)

    h2 = _make_tc_mid(N, NPAD, F1, F2)(
        part1, h1, inv2, b1.reshape(1, F1), g1.reshape(1, F1),
        be1.reshape(1, F1), W2)
    part2 = _make_sc_agg(E, NPAD, F2)(
        h2.reshape(NPAD * (F2 // 8), 8), src, dst, norm)

    z, recon = _make_tc_post(N, NPAD, F2, FD1, DOUT)(
        part2, h2, inv2, b2.reshape(1, F2), g2.reshape(1, F2),
        be2.reshape(1, F2), Wd1, bd1.reshape(1, FD1), gd1.reshape(1, FD1),
        bed1.reshape(1, FD1), Wd2, bd2.reshape(1, DOUT))
    return (recon, z)


# speedup vs baseline: 18.5934x; 1.4398x over previous
import functools

import jax
import jax.numpy as jnp
from jax import lax
from jax.experimental import pallas as pl
from jax.experimental.pallas import tpu as pltpu
from jax.experimental.pallas import tpu_sc as plsc

NLANE = 16
NTILE = 16
NCORE = 2
NW = NCORE * NTILE


def _mesh():
    return plsc.VectorSubcoreMesh(core_axis_name="c", subcore_axis_name="s")


_SC_PARAMS = dict(needs_layout_passes=False, use_tc_tiling_on_sc=False)


@functools.lru_cache(maxsize=None)
def _make_sc_norm(E, NPAD):
    SLICE = NPAD // NTILE
    ED = E // NTILE
    ECH = 2000
    EN = E // NW
    NCH = 2000
    assert ED % ECH == 0 and EN % NCH == 0 and SLICE % NLANE == 0

    def body(src_hbm, dst_hbm, ew_hbm, norm_hbm, inv_hbm,
             acc, dis_loc, sb, db, wb, nb, psum, dslice, islice,
             sp_part, sp_dis):
        cid = lax.axis_index("c")
        sid = lax.axis_index("s")
        fzero = jnp.zeros((NLANE,), jnp.float32)

        def zloop(i, c):
            for u in range(8):
                acc[pl.ds((8 * i + u) * NLANE, NLANE)] = fzero
            return c
        lax.fori_loop(0, NPAD // (8 * NLANE), zloop, 0)

        def deg_chunk(k, c):
            base = sid * ED + k * ECH
            pltpu.sync_copy(dst_hbm.at[pl.ds(base, ECH)], db)
            pltpu.sync_copy(ew_hbm.at[pl.ds(base, ECH)], wb)

            def deg16(i, cc):
                for u in range(5):
                    sl = pl.ds((5 * i + u) * NLANE, NLANE)
                    plsc.addupdate_scatter(acc, [db[sl]], wb[sl])
                return cc
            lax.fori_loop(0, ECH // (5 * NLANE), deg16, 0)
            return c
        lax.fori_loop(0, ED // ECH, deg_chunk, 0)

        pltpu.sync_copy(acc, sp_part.at[pl.ds(sid * NPAD, NPAD)])
        plsc.subcore_barrier()

        for r in range(NTILE):
            pltpu.sync_copy(sp_part.at[pl.ds(r * NPAD + sid * SLICE, SLICE)],
                            psum.at[pl.ds(r * SLICE, SLICE)])

        magic = jnp.full((NLANE,), 0x5F3759DF, jnp.int32)

        def disloop(i, c):
            s = psum[pl.ds(i * NLANE, NLANE)]
            for r in range(1, NTILE):
                s = s + psum[pl.ds(r * SLICE + i * NLANE, NLANE)]
            d = s + 1.0
            ibits = plsc.bitcast(d, jnp.int32)
            y = plsc.bitcast(magic - lax.shift_right_logical(ibits, 1),
                             jnp.float32)
            for _ in range(3):
                y = y * (1.5 - 0.5 * d * y * y)
            dslice[pl.ds(i * NLANE, NLANE)] = y
            islice[pl.ds(i * NLANE, NLANE)] = y * y
            return c
        lax.fori_loop(0, SLICE // NLANE, disloop, 0)

        pltpu.sync_copy(dslice, sp_dis.at[pl.ds(sid * SLICE, SLICE)])

        @pl.when(cid == 0)
        def _():
            pltpu.sync_copy(islice, inv_hbm.at[pl.ds(sid * SLICE, SLICE)])

        plsc.subcore_barrier()
        pltpu.sync_copy(sp_dis, dis_loc)

        wid = cid * NTILE + sid

        def nchunk(k, c):
            base = wid * EN + k * NCH
            pltpu.sync_copy(src_hbm.at[pl.ds(base, NCH)], sb)
            pltpu.sync_copy(dst_hbm.at[pl.ds(base, NCH)], db)
            pltpu.sync_copy(ew_hbm.at[pl.ds(base, NCH)], wb)

            def n16(i, cc):
                for u in range(5):
                    sl = pl.ds((5 * i + u) * NLANE, NLANE)
                    a = plsc.load_gather(dis_loc, [sb[sl]])
                    b = plsc.load_gather(dis_loc, [db[sl]])
                    nb[sl] = a * wb[sl] * b
                return cc
            lax.fori_loop(0, NCH // (5 * NLANE), n16, 0)
            pltpu.sync_copy(nb, norm_hbm.at[pl.ds(base, NCH)])
            return c
        lax.fori_loop(0, EN // NCH, nchunk, 0)

    return pl.kernel(
        body,
        out_type=[jax.ShapeDtypeStruct((E,), jnp.float32),
                  jax.ShapeDtypeStruct((NPAD,), jnp.float32)],
        mesh=_mesh(),
        compiler_params=pltpu.CompilerParams(**_SC_PARAMS),
        scratch_types=[
            pltpu.VMEM((NPAD,), jnp.float32),
            pltpu.VMEM((NPAD,), jnp.float32),
            pltpu.VMEM((NCH,), jnp.int32),
            pltpu.VMEM((NCH,), jnp.int32),
            pltpu.VMEM((NCH,), jnp.float32),
            pltpu.VMEM((NCH,), jnp.float32),
            pltpu.VMEM((NTILE * SLICE,), jnp.float32),
            pltpu.VMEM((SLICE,), jnp.float32),
            pltpu.VMEM((SLICE,), jnp.float32),
            pltpu.VMEM_SHARED((NTILE * NPAD,), jnp.float32),
            pltpu.VMEM_SHARED((NPAD,), jnp.float32),
        ],
    )


@functools.lru_cache(maxsize=None)
def _make_sc_agg(E, NPAD, F):
    P = F // 8
    GPS = NTILE // P
    NG = NCORE * GPS
    EG = E // NG
    NC = 50
    CH = EG // NC
    SLEN = 400
    NS = CH // SLEN
    NROWS = NPAD // GPS
    RB = 640
    UN = 8
    assert EG % NC == 0 and CH % SLEN == 0 and NROWS % RB == 0
    assert NC % 2 == 0 and (CH // 2) % UN == 0 and CH % (2 * NLANE) == 0

    def body(tab_hbm, src_hbm, dst_hbm, norm_hbm, out_hbm,
             acc, rows0, rows1, rbuf, sb0, sb1, db0, db1, nb0, nb1,
             sp_stage, seml0, seml1, semr0, semr1):
        rows = (rows0, rows1)
        sb = (sb0, sb1)
        db = (db0, db1)
        nb = (nb0, nb1)
        seml = (seml0, seml1)
        semr = (semr0, semr1)
        cid = lax.axis_index("c")
        sid = lax.axis_index("s")
        gl = sid // P
        t = sid - gl * P
        g = cid * GPS + gl
        iota = lax.iota(jnp.int32, NLANE)
        col = jnp.bitwise_and(iota, 7)
        c01 = lax.shift_right_logical(iota, 3)
        c01u = [c01 + 2 * u for u in range(UN)]
        mlow = iota < 8
        mhigh = jnp.logical_not(mlow)
        fzero = jnp.zeros((NLANE,), jnp.float32)
        two = jnp.full((NLANE,), 2, jnp.int32)

        def zloop(i, c):
            for u in range(8):
                plsc.store_scatter(acc, [two * (8 * i + u) + c01, col],
                                   fzero)
            return c
        lax.fori_loop(0, NPAD // 16, zloop, 0)

        toff = jnp.full((NLANE,), 1, jnp.int32) * t

        def lin_issue(k, s):
            base = g * EG + k * CH
            pltpu.async_copy(src_hbm.at[pl.ds(base, CH)], sb[s], seml[s])
            pltpu.async_copy(dst_hbm.at[pl.ds(base, CH)], db[s], seml[s])
            pltpu.async_copy(norm_hbm.at[pl.ds(base, CH)], nb[s], seml[s])

        def lin_wait(s):
            pltpu.make_async_copy(src_hbm.at[pl.ds(0, CH)], sb[s],
                                  seml[s]).wait()
            pltpu.make_async_copy(dst_hbm.at[pl.ds(0, CH)], db[s],
                                  seml[s]).wait()
            pltpu.make_async_copy(norm_hbm.at[pl.ds(0, CH)], nb[s],
                                  seml[s]).wait()

        def adj(s):
            def adj1(i, cc):
                for u in range(2):
                    sl = pl.ds((2 * i + u) * NLANE, NLANE)
                    sb[s][sl] = sb[s][sl] * P + toff
                return cc
            lax.fori_loop(0, CH // (2 * NLANE), adj1, 0)

        def rows_fire(s):
            for j in range(NS):
                pltpu.async_copy(
                    tab_hbm.at[sb[s].at[pl.ds(j * SLEN, SLEN)]],
                    rows[s].at[pl.ds(j * SLEN, SLEN)], semr[s])

        def rows_wait(s):
            for j in range(NS):
                pltpu.make_async_copy(
                    tab_hbm.at[sb[s].at[pl.ds(j * SLEN, SLEN)]],
                    rows[s].at[pl.ds(j * SLEN, SLEN)], semr[s]).wait()

        def compute(s):
            @plsc.parallel_loop(0, CH // (2 * UN), step=1, unroll=2)
            def pairs(q):
                base = q * (2 * UN)
                d16 = db[s][pl.ds(base, 2 * UN)]
                n16 = nb[s][pl.ds(base, 2 * UN)]
                b16 = jnp.full((NLANE,), 2 * UN, jnp.int32) * q
                for u in range(UN):
                    dstp = d16.at[c01u[u]].get(mode="promise_in_bounds")
                    normp = n16.at[c01u[u]].get(mode="promise_in_bounds")
                    psel = b16 + c01u[u]
                    r16 = plsc.load_gather(rows[s], [psel, col])
                    val = r16 * normp
                    plsc.addupdate_scatter(acc, [dstp, col], val)

        lin_issue(0, 0)
        lin_wait(0)
        adj(0)
        rows_fire(0)
        lin_issue(1, 1)

        def piter(ko, c):
            for u2 in range(2):
                k = ko * 2 + u2
                s = u2
                rows_wait(s)

                @pl.when(k + 1 < NC)
                def _():
                    lin_wait(1 - s)
                    adj(1 - s)
                    rows_fire(1 - s)

                compute(s)

                @pl.when(k + 2 < NC)
                def _():
                    lin_issue(k + 2, s)
            return c
        lax.fori_loop(0, NC // 2, piter, 0)

        qbase = gl * NROWS
        for o in range(1, GPS):
            src_tile = ((gl + o) % GPS) * P + t
            give = ((gl + GPS - o) % GPS) * NROWS

            def fold_blk(b, c):
                pltpu.sync_copy(acc.at[pl.ds(give + b * RB, RB)],
                                sp_stage.at[sid])
                plsc.subcore_barrier()
                pltpu.sync_copy(sp_stage.at[src_tile], rbuf)
                rowb = qbase + b * RB

                def fold16(i, cc):
                    for u in range(8):
                        rp = two * (8 * i + u) + c01
                        v = plsc.load_gather(rbuf, [rp, col])
                        plsc.addupdate_scatter(acc, [rp + rowb, col], v)
                    return cc
                lax.fori_loop(0, RB // 16, fold16, 0)
                plsc.subcore_barrier()
                return c
            lax.fori_loop(0, NROWS // RB, fold_blk, 0)

        pltpu.sync_copy(acc.at[pl.ds(qbase, NROWS)],
                        out_hbm.at[cid, pl.ds(qbase, NROWS),
                                   pl.ds(t * 8, 8)])

    return pl.kernel(
        body,
        out_type=jax.ShapeDtypeStruct((NCORE, NPAD, F), jnp.float32),
        mesh=_mesh(),
        compiler_params=pltpu.CompilerParams(**_SC_PARAMS),
        scratch_types=[
            pltpu.VMEM((NPAD, 8), jnp.float32),
            pltpu.VMEM((CH, 8), jnp.float32),
            pltpu.VMEM((CH, 8), jnp.float32),
            pltpu.VMEM((RB, 8), jnp.float32),
            pltpu.VMEM((CH,), jnp.int32),
            pltpu.VMEM((CH,), jnp.int32),
            pltpu.VMEM((CH,), jnp.int32),
            pltpu.VMEM((CH,), jnp.int32),
            pltpu.VMEM((CH,), jnp.float32),
            pltpu.VMEM((CH,), jnp.float32),
            pltpu.VMEM_SHARED((NTILE, RB, 8), jnp.float32),
            pltpu.SemaphoreType.DMA,
            pltpu.SemaphoreType.DMA,
            pltpu.SemaphoreType.DMA,
            pltpu.SemaphoreType.DMA,
        ],
    )


@functools.lru_cache(maxsize=None)
def _make_tc_pre(N, NPAD, DIN, F):
    def body(x_ref, w_ref, out_ref):
        h = jnp.dot(x_ref[...], w_ref[...],
                    preferred_element_type=jnp.float32)
        out_ref[:N] = h
        out_ref[N:] = jnp.zeros((NPAD - N, F), jnp.float32)

    return pl.pallas_call(
        body,
        out_shape=jax.ShapeDtypeStruct((NPAD, F), jnp.float32))


def _bn(h, g, b):
    m = jnp.mean(h, axis=0, keepdims=True)
    v = jnp.mean((h - m) * (h - m), axis=0, keepdims=True)
    return (h - m) * lax.rsqrt(v + 1e-5) * g + b


@functools.lru_cache(maxsize=None)
def _make_tc_mid(N, NPAD, F1, F2):
    def body(part_ref, ht_ref, inv_ref, b_ref, g_ref, be_ref, w_ref,
             out_ref):
        inv = inv_ref[:N]
        h = (part_ref[0, :N, :] + part_ref[1, :N, :]
             + inv * ht_ref[:N, :] + b_ref[0])
        h = jnp.maximum(_bn(h, g_ref[0], be_ref[0]), 0.0)
        h2 = jnp.dot(h, w_ref[...], preferred_element_type=jnp.float32)
        out_ref[:N] = h2
        out_ref[N:] = jnp.zeros((NPAD - N, F2), jnp.float32)

    return pl.pallas_call(
        body,
        out_shape=jax.ShapeDtypeStruct((NPAD, F2), jnp.float32))


@functools.lru_cache(maxsize=None)
def _make_tc_post(N, NPAD, F2, FD1, DOUT):
    def body(part_ref, ht_ref, inv_ref, b2_ref, g2_ref, be2_ref,
             wd1_ref, bd1_ref, gd1_ref, bed1_ref, wd2_ref, bd2_ref,
             z_ref, recon_ref):
        inv = inv_ref[:N]
        h = (part_ref[0, :N, :] + part_ref[1, :N, :]
             + inv * ht_ref[:N, :] + b2_ref[0])
        z = jnp.maximum(_bn(h, g2_ref[0], be2_ref[0]), 0.0)
        z_ref[...] = z
        d = jnp.dot(z, wd1_ref[...],
                    preferred_element_type=jnp.float32) + bd1_ref[0]
        d = jnp.maximum(_bn(d, gd1_ref[0], bed1_ref[0]), 0.0)
        recon_ref[...] = jnp.dot(d, wd2_ref[...],
                                 preferred_element_type=jnp.float32) + bd2_ref[0]

    return pl.pallas_call(
        body,
        out_shape=[jax.ShapeDtypeStruct((N, F2), jnp.float32),
                   jax.ShapeDtypeStruct((N, DOUT), jnp.float32)])


def kernel(x, edge_index, edge_attr, W1, b1, g1, be1, W2, b2, g2, be2,
           Wd1, bd1, gd1, bed1, Wd2, bd2):
    N, DIN = x.shape
    E = edge_attr.shape[0]
    F1, F2 = W1.shape[1], W2.shape[1]
    FD1, DOUT = Wd1.shape[1], Wd2.shape[1]
    NPAD = ((N + 255) // 256) * 256

    src = edge_index[0]
    dst = edge_index[1]

    norm, invp = _make_sc_norm(E, NPAD)(src, dst, edge_attr)
    inv2 = invp.reshape(NPAD, 1)

    h1 = _make_tc_pre(N, NPAD, DIN, F1)(x, W1)
    part1 = _make_sc_agg(E, NPAD, F1)(
        h1.reshape(NPAD * (F1 // 8), 8), src, dst, norm)

    h2 = _make_tc_mid(N, NPAD, F1, F2)(
        part1, h1, inv2, b1.reshape(1, F1), g1.reshape(1, F1),
        be1.reshape(1, F1), W2)
    part2 = _make_sc_agg(E, NPAD, F2)(
        h2.reshape(NPAD * (F2 // 8), 8), src, dst, norm)

    z, recon = _make_tc_post(N, NPAD, F2, FD1, DOUT)(
        part2, h2, inv2, b2.reshape(1, F2), g2.reshape(1, F2),
        be2.reshape(1, F2), Wd1, bd1.reshape(1, FD1), gd1.reshape(1, FD1),
        bed1.reshape(1, FD1), Wd2, bd2.reshape(1, DOUT))
    return (recon, z)


# speedup vs baseline: 20.0416x; 1.0779x over previous
import functools

import jax
import jax.numpy as jnp
from jax import lax
from jax.experimental import pallas as pl
from jax.experimental.pallas import tpu as pltpu
from jax.experimental.pallas import tpu_sc as plsc

NLANE = 16
NTILE = 16
NCORE = 2
NW = NCORE * NTILE


def _mesh():
    return plsc.VectorSubcoreMesh(core_axis_name="c", subcore_axis_name="s")


_SC_PARAMS = dict(needs_layout_passes=False, use_tc_tiling_on_sc=False)


@functools.lru_cache(maxsize=None)
def _make_sc_norm(E, NPAD):
    SLICE = NPAD // NTILE
    ED = E // NTILE
    ECH = 2000
    EN = E // NW
    NCH = 2000
    assert ED % ECH == 0 and EN % NCH == 0 and SLICE % NLANE == 0

    def body(src_hbm, dst_hbm, ew_hbm, norm_hbm, inv_hbm,
             acc, dis_loc, sb, db, wb, nb, psum, dslice, islice,
             sp_part, sp_dis):
        cid = lax.axis_index("c")
        sid = lax.axis_index("s")
        fzero = jnp.zeros((NLANE,), jnp.float32)

        @plsc.parallel_loop(0, NPAD // (8 * NLANE), step=1, unroll=2)
        def zloop(i):
            for u in range(8):
                acc[pl.ds((8 * i + u) * NLANE, NLANE)] = fzero

        def deg_chunk(k, c):
            base = sid * ED + k * ECH
            pltpu.sync_copy(dst_hbm.at[pl.ds(base, ECH)], db)
            pltpu.sync_copy(ew_hbm.at[pl.ds(base, ECH)], wb)

            @plsc.parallel_loop(0, ECH // (5 * NLANE), step=1, unroll=2)
            def deg16(i):
                for u in range(5):
                    sl = pl.ds((5 * i + u) * NLANE, NLANE)
                    plsc.addupdate_scatter(acc, [db[sl]], wb[sl])
            return c
        lax.fori_loop(0, ED // ECH, deg_chunk, 0)

        pltpu.sync_copy(acc, sp_part.at[pl.ds(sid * NPAD, NPAD)])
        plsc.subcore_barrier()

        for r in range(NTILE):
            pltpu.sync_copy(sp_part.at[pl.ds(r * NPAD + sid * SLICE, SLICE)],
                            psum.at[pl.ds(r * SLICE, SLICE)])

        magic = jnp.full((NLANE,), 0x5F3759DF, jnp.int32)

        def disloop(i, c):
            s = psum[pl.ds(i * NLANE, NLANE)]
            for r in range(1, NTILE):
                s = s + psum[pl.ds(r * SLICE + i * NLANE, NLANE)]
            d = s + 1.0
            ibits = plsc.bitcast(d, jnp.int32)
            y = plsc.bitcast(magic - lax.shift_right_logical(ibits, 1),
                             jnp.float32)
            for _ in range(3):
                y = y * (1.5 - 0.5 * d * y * y)
            dslice[pl.ds(i * NLANE, NLANE)] = y
            islice[pl.ds(i * NLANE, NLANE)] = y * y
            return c
        lax.fori_loop(0, SLICE // NLANE, disloop, 0)

        pltpu.sync_copy(dslice, sp_dis.at[pl.ds(sid * SLICE, SLICE)])

        @pl.when(cid == 0)
        def _():
            pltpu.sync_copy(islice, inv_hbm.at[pl.ds(sid * SLICE, SLICE)])

        plsc.subcore_barrier()
        pltpu.sync_copy(sp_dis, dis_loc)

        wid = cid * NTILE + sid

        def nchunk(k, c):
            base = wid * EN + k * NCH
            pltpu.sync_copy(src_hbm.at[pl.ds(base, NCH)], sb)
            pltpu.sync_copy(dst_hbm.at[pl.ds(base, NCH)], db)
            pltpu.sync_copy(ew_hbm.at[pl.ds(base, NCH)], wb)

            @plsc.parallel_loop(0, NCH // (5 * NLANE), step=1, unroll=2)
            def n16(i):
                for u in range(5):
                    sl = pl.ds((5 * i + u) * NLANE, NLANE)
                    a = plsc.load_gather(dis_loc, [sb[sl]])
                    b = plsc.load_gather(dis_loc, [db[sl]])
                    nb[sl] = a * wb[sl] * b
            pltpu.sync_copy(nb, norm_hbm.at[pl.ds(base, NCH)])
            return c
        lax.fori_loop(0, EN // NCH, nchunk, 0)

    return pl.kernel(
        body,
        out_type=[jax.ShapeDtypeStruct((E,), jnp.float32),
                  jax.ShapeDtypeStruct((NPAD,), jnp.float32)],
        mesh=_mesh(),
        compiler_params=pltpu.CompilerParams(**_SC_PARAMS),
        scratch_types=[
            pltpu.VMEM((NPAD,), jnp.float32),
            pltpu.VMEM((NPAD,), jnp.float32),
            pltpu.VMEM((NCH,), jnp.int32),
            pltpu.VMEM((NCH,), jnp.int32),
            pltpu.VMEM((NCH,), jnp.float32),
            pltpu.VMEM((NCH,), jnp.float32),
            pltpu.VMEM((NTILE * SLICE,), jnp.float32),
            pltpu.VMEM((SLICE,), jnp.float32),
            pltpu.VMEM((SLICE,), jnp.float32),
            pltpu.VMEM_SHARED((NTILE * NPAD,), jnp.float32),
            pltpu.VMEM_SHARED((NPAD,), jnp.float32),
        ],
    )


@functools.lru_cache(maxsize=None)
def _make_sc_agg(E, NPAD, F):
    P = F // 8
    GPS = NTILE // P
    NG = NCORE * GPS
    EG = E // NG
    NC = 50
    CH = EG // NC
    SLEN = 400
    NS = CH // SLEN
    NROWS = NPAD // GPS
    RB = 640
    UN = 8
    assert EG % NC == 0 and CH % SLEN == 0 and NROWS % RB == 0
    assert NC % 2 == 0 and (CH // 2) % UN == 0 and CH % (2 * NLANE) == 0

    def body(tab_hbm, src_hbm, dst_hbm, norm_hbm, out_hbm,
             acc, rows0, rows1, rbuf, sb0, sb1, db0, db1, nb0, nb1,
             sp_stage, seml0, seml1, semr0, semr1):
        rows = (rows0, rows1)
        sb = (sb0, sb1)
        db = (db0, db1)
        nb = (nb0, nb1)
        seml = (seml0, seml1)
        semr = (semr0, semr1)
        cid = lax.axis_index("c")
        sid = lax.axis_index("s")
        gl = sid // P
        t = sid - gl * P
        g = cid * GPS + gl
        iota = lax.iota(jnp.int32, NLANE)
        col = jnp.bitwise_and(iota, 7)
        c01 = lax.shift_right_logical(iota, 3)
        c01u = [c01 + 2 * u for u in range(UN)]
        mlow = iota < 8
        mhigh = jnp.logical_not(mlow)
        fzero = jnp.zeros((NLANE,), jnp.float32)
        two = jnp.full((NLANE,), 2, jnp.int32)

        @plsc.parallel_loop(0, NPAD // 16, step=1, unroll=2)
        def zloop(i):
            for u in range(8):
                plsc.store_scatter(acc, [two * (8 * i + u) + c01, col],
                                   fzero)

        toff = jnp.full((NLANE,), 1, jnp.int32) * t

        def lin_issue(k, s):
            base = g * EG + k * CH
            pltpu.async_copy(src_hbm.at[pl.ds(base, CH)], sb[s], seml[s])
            pltpu.async_copy(dst_hbm.at[pl.ds(base, CH)], db[s], seml[s])
            pltpu.async_copy(norm_hbm.at[pl.ds(base, CH)], nb[s], seml[s])

        def lin_wait(s):
            pltpu.make_async_copy(src_hbm.at[pl.ds(0, CH)], sb[s],
                                  seml[s]).wait()
            pltpu.make_async_copy(dst_hbm.at[pl.ds(0, CH)], db[s],
                                  seml[s]).wait()
            pltpu.make_async_copy(norm_hbm.at[pl.ds(0, CH)], nb[s],
                                  seml[s]).wait()

        def adj(s):
            @plsc.parallel_loop(0, CH // (2 * NLANE), step=1, unroll=2)
            def adj1(i):
                for u in range(2):
                    sl = pl.ds((2 * i + u) * NLANE, NLANE)
                    sb[s][sl] = sb[s][sl] * P + toff

        def rows_fire(s):
            for j in range(NS):
                pltpu.async_copy(
                    tab_hbm.at[sb[s].at[pl.ds(j * SLEN, SLEN)]],
                    rows[s].at[pl.ds(j * SLEN, SLEN)], semr[s])

        def rows_wait(s):
            for j in range(NS):
                pltpu.make_async_copy(
                    tab_hbm.at[sb[s].at[pl.ds(j * SLEN, SLEN)]],
                    rows[s].at[pl.ds(j * SLEN, SLEN)], semr[s]).wait()

        def compute(s):
            @plsc.parallel_loop(0, CH // (2 * UN), step=1, unroll=2)
            def pairs(q):
                base = q * (2 * UN)
                d16 = db[s][pl.ds(base, 2 * UN)]
                n16 = nb[s][pl.ds(base, 2 * UN)]
                b16 = jnp.full((NLANE,), 2 * UN, jnp.int32) * q
                for u in range(UN):
                    dstp = d16.at[c01u[u]].get(mode="promise_in_bounds")
                    normp = n16.at[c01u[u]].get(mode="promise_in_bounds")
                    psel = b16 + c01u[u]
                    r16 = plsc.load_gather(rows[s], [psel, col])
                    val = r16 * normp
                    plsc.addupdate_scatter(acc, [dstp, col], val)

        lin_issue(0, 0)
        lin_wait(0)
        adj(0)
        rows_fire(0)
        lin_issue(1, 1)

        def piter(ko, c):
            for u2 in range(2):
                k = ko * 2 + u2
                s = u2
                rows_wait(s)

                @pl.when(k + 1 < NC)
                def _():
                    lin_wait(1 - s)
                    adj(1 - s)
                    rows_fire(1 - s)

                compute(s)

                @pl.when(k + 2 < NC)
                def _():
                    lin_issue(k + 2, s)
            return c
        lax.fori_loop(0, NC // 2, piter, 0)

        qbase = gl * NROWS
        for o in range(1, GPS):
            src_tile = ((gl + o) % GPS) * P + t
            give = ((gl + GPS - o) % GPS) * NROWS

            def fold_blk(b, c):
                pltpu.sync_copy(acc.at[pl.ds(give + b * RB, RB)],
                                sp_stage.at[sid])
                plsc.subcore_barrier()
                pltpu.sync_copy(sp_stage.at[src_tile], rbuf)
                rowb = qbase + b * RB

                @plsc.parallel_loop(0, RB // 16, step=1, unroll=2)
                def fold16(i):
                    for u in range(8):
                        rp = two * (8 * i + u) + c01
                        v = plsc.load_gather(rbuf, [rp, col])
                        plsc.addupdate_scatter(acc, [rp + rowb, col], v)
                plsc.subcore_barrier()
                return c
            lax.fori_loop(0, NROWS // RB, fold_blk, 0)

        pltpu.sync_copy(acc.at[pl.ds(qbase, NROWS)],
                        out_hbm.at[cid, pl.ds(qbase, NROWS),
                                   pl.ds(t * 8, 8)])

    return pl.kernel(
        body,
        out_type=jax.ShapeDtypeStruct((NCORE, NPAD, F), jnp.float32),
        mesh=_mesh(),
        compiler_params=pltpu.CompilerParams(**_SC_PARAMS),
        scratch_types=[
            pltpu.VMEM((NPAD, 8), jnp.float32),
            pltpu.VMEM((CH, 8), jnp.float32),
            pltpu.VMEM((CH, 8), jnp.float32),
            pltpu.VMEM((RB, 8), jnp.float32),
            pltpu.VMEM((CH,), jnp.int32),
            pltpu.VMEM((CH,), jnp.int32),
            pltpu.VMEM((CH,), jnp.int32),
            pltpu.VMEM((CH,), jnp.int32),
            pltpu.VMEM((CH,), jnp.float32),
            pltpu.VMEM((CH,), jnp.float32),
            pltpu.VMEM_SHARED((NTILE, RB, 8), jnp.float32),
            pltpu.SemaphoreType.DMA,
            pltpu.SemaphoreType.DMA,
            pltpu.SemaphoreType.DMA,
            pltpu.SemaphoreType.DMA,
        ],
    )


@functools.lru_cache(maxsize=None)
def _make_tc_pre(N, NPAD, DIN, F):
    def body(x_ref, w_ref, out_ref):
        h = jnp.dot(x_ref[...], w_ref[...],
                    preferred_element_type=jnp.float32)
        out_ref[:N] = h
        out_ref[N:] = jnp.zeros((NPAD - N, F), jnp.float32)

    return pl.pallas_call(
        body,
        out_shape=jax.ShapeDtypeStruct((NPAD, F), jnp.float32))


def _bn(h, g, b):
    m = jnp.mean(h, axis=0, keepdims=True)
    v = jnp.mean((h - m) * (h - m), axis=0, keepdims=True)
    return (h - m) * lax.rsqrt(v + 1e-5) * g + b


@functools.lru_cache(maxsize=None)
def _make_tc_mid(N, NPAD, F1, F2):
    def body(part_ref, ht_ref, inv_ref, b_ref, g_ref, be_ref, w_ref,
             out_ref):
        inv = inv_ref[:N]
        h = (part_ref[0, :N, :] + part_ref[1, :N, :]
             + inv * ht_ref[:N, :] + b_ref[0])
        h = jnp.maximum(_bn(h, g_ref[0], be_ref[0]), 0.0)
        h2 = jnp.dot(h, w_ref[...], preferred_element_type=jnp.float32)
        out_ref[:N] = h2
        out_ref[N:] = jnp.zeros((NPAD - N, F2), jnp.float32)

    return pl.pallas_call(
        body,
        out_shape=jax.ShapeDtypeStruct((NPAD, F2), jnp.float32))


@functools.lru_cache(maxsize=None)
def _make_tc_post(N, NPAD, F2, FD1, DOUT):
    def body(part_ref, ht_ref, inv_ref, b2_ref, g2_ref, be2_ref,
             wd1_ref, bd1_ref, gd1_ref, bed1_ref, wd2_ref, bd2_ref,
             z_ref, recon_ref):
        inv = inv_ref[:N]
        h = (part_ref[0, :N, :] + part_ref[1, :N, :]
             + inv * ht_ref[:N, :] + b2_ref[0])
        z = jnp.maximum(_bn(h, g2_ref[0], be2_ref[0]), 0.0)
        z_ref[...] = z
        d = jnp.dot(z, wd1_ref[...],
                    preferred_element_type=jnp.float32) + bd1_ref[0]
        d = jnp.maximum(_bn(d, gd1_ref[0], bed1_ref[0]), 0.0)
        recon_ref[...] = jnp.dot(d, wd2_ref[...],
                                 preferred_element_type=jnp.float32) + bd2_ref[0]

    return pl.pallas_call(
        body,
        out_shape=[jax.ShapeDtypeStruct((N, F2), jnp.float32),
                   jax.ShapeDtypeStruct((N, DOUT), jnp.float32)])


def kernel(x, edge_index, edge_attr, W1, b1, g1, be1, W2, b2, g2, be2,
           Wd1, bd1, gd1, bed1, Wd2, bd2):
    N, DIN = x.shape
    E = edge_attr.shape[0]
    F1, F2 = W1.shape[1], W2.shape[1]
    FD1, DOUT = Wd1.shape[1], Wd2.shape[1]
    NPAD = ((N + 255) // 256) * 256

    src = edge_index[0]
    dst = edge_index[1]

    norm, invp = _make_sc_norm(E, NPAD)(src, dst, edge_attr)
    inv2 = invp.reshape(NPAD, 1)

    h1 = _make_tc_pre(N, NPAD, DIN, F1)(x, W1)
    part1 = _make_sc_agg(E, NPAD, F1)(
        h1.reshape(NPAD * (F1 // 8), 8), src, dst, norm)

    h2 = _make_tc_mid(N, NPAD, F1, F2)(
        part1, h1, inv2, b1.reshape(1, F1), g1.reshape(1, F1),
        be1.reshape(1, F1), W2)
    part2 = _make_sc_agg(E, NPAD, F2)(
        h2.reshape(NPAD * (F2 // 8), 8), src, dst, norm)

    z, recon = _make_tc_post(N, NPAD, F2, FD1, DOUT)(
        part2, h2, inv2, b2.reshape(1, F2), g2.reshape(1, F2),
        be2.reshape(1, F2), Wd1, bd1.reshape(1, FD1), gd1.reshape(1, FD1),
        bed1.reshape(1, FD1), Wd2, bd2.reshape(1, DOUT))
    return (recon, z)
